# baseline (speedup 1.0000x reference)
import jax
import jax.numpy as jnp
from jax.experimental import pallas as pl

N = 100000
E = 1600000
DIN = 17
H = 64
H2 = 128
OUT = 2
NEG = 0.05
EPS = 1e-5

_BLK = 4000


def _ln(x, g, b):
    m = jnp.mean(x, axis=-1, keepdims=True)
    v = jnp.var(x, axis=-1, keepdims=True)
    return (x - m) / jnp.sqrt(v + EPS) * g + b


def _pre_body(h_ref, win_ref, bin_ref, wt1_ref, bt1_ref, wt2_ref, bt2_ref,
              wg_ref, bg_ref, x_ref, g_ref):
    x = jnp.dot(h_ref[...], win_ref[...], preferred_element_type=jnp.float32)
    x = x + bin_ref[...]
    x = jnp.dot(x, wt1_ref[...], preferred_element_type=jnp.float32) + bt1_ref[...]
    x = jnp.where(x >= 0, x, NEG * x)
    x = jnp.dot(x, wt2_ref[...], preferred_element_type=jnp.float32) + bt2_ref[...]
    x_ref[...] = x
    g_ref[...] = jnp.dot(x, wg_ref[...], preferred_element_type=jnp.float32) + bg_ref[...]


def _pre_chain(h, p):
    grid = (N // _BLK,)
    full = lambda shape: pl.BlockSpec(shape, lambda i: (0, 0))
    return pl.pallas_call(
        _pre_body,
        grid=grid,
        in_specs=[
            pl.BlockSpec((_BLK, DIN), lambda i: (i, 0)),
            full((DIN, H)), full((1, H)),
            full((H, H)), full((1, H)),
            full((H, H)), full((1, H)),
            full((H, 1)), full((1, 1)),
        ],
        out_specs=[
            pl.BlockSpec((_BLK, H), lambda i: (i, 0)),
            pl.BlockSpec((_BLK, 1), lambda i: (i, 0)),
        ],
        out_shape=[
            jax.ShapeDtypeStruct((N, H), jnp.float32),
            jax.ShapeDtypeStruct((N, 1), jnp.float32),
        ],
    )(h, p['W_in'], p['b_in'][None], p['W_t1'], p['b_t1'][None],
      p['W_t2'], p['b_t2'][None], p['W_gate'], p['b_gate'][None])


def kernel(h, params, edge_index):
    src = edge_index[0]
    dst = edge_index[1]
    p = params
    x, gate = _pre_chain(h, p)

    deg_out = jnp.maximum(jnp.zeros((N,), jnp.float32).at[src].add(1.0), 1.0)
    deg_in = jnp.maximum(jnp.zeros((N,), jnp.float32).at[dst].add(1.0), 1.0)
    xs = x * (deg_out ** -0.5)[:, None]
    agg = jnp.zeros_like(x).at[dst].add(xs[src])
    agg = agg * (deg_in ** -0.5)[:, None]
    gcn_h = _ln(agg @ p['W_gcn'] + p['b_gcn'], p['ln_gcn_g'], p['ln_gcn_b'])
    loc_h = gcn_h - x
    a = jax.nn.softmax(gate, axis=0)
    pool = jnp.sum(a * x, axis=0, keepdims=True)
    glo_h = jnp.broadcast_to(pool, x.shape) - x
    Z = jnp.concatenate([loc_h, glo_h], axis=1)
    hh = Z
    for lp in p['layers']:
        hp = jax.nn.relu(hh @ lp['Wp'] + lp['bp'])
        neigh = jax.ops.segment_max(hp[src], dst, num_segments=N)
        neigh = jnp.where(jnp.isfinite(neigh), neigh, 0.0)
        out = hh @ lp['Ws'] + neigh @ lp['Wn'] + lp['bs']
        hh = jax.nn.relu(_ln(out, lp['ln_g'], lp['ln_b']))
    score = hh @ p['W_out'] + p['b_out']
    return score, Z


# speedup vs baseline: 1.4808x; 1.4808x over previous
import functools

import jax
import jax.numpy as jnp
from jax import lax
from jax.experimental import pallas as pl
from jax.experimental.pallas import tpu as pltpu
from jax.experimental.pallas import tpu_sc as plsc

N = 100000
E = 1600000
DIN = 17
H = 64
H2 = 128
OUT = 2
NEG = 0.05
EPS = 1e-5

NC, NS, LANES = 2, 16, 16
N_PAD = 100096
NPT = N_PAD // NS
ZR = NPT // 8
EW = 128
ROWS = 12544
E_PAD = ROWS * EW
RPT = ROWS // NS
KB = 4
NBLK = RPT // KB
CZ = 391
NCZ = NPT // CZ

_BLK = 4000


def _ln(x, g, b):
    m = jnp.mean(x, axis=-1, keepdims=True)
    v = jnp.var(x, axis=-1, keepdims=True)
    return (x - m) / jnp.sqrt(v + EPS) * g + b


def _pre_body(h_ref, win_ref, bin_ref, wt1_ref, bt1_ref, wt2_ref, bt2_ref,
              wg_ref, bg_ref, x_ref, g_ref):
    x = jnp.dot(h_ref[...], win_ref[...], preferred_element_type=jnp.float32)
    x = x + bin_ref[...]
    x = jnp.dot(x, wt1_ref[...], preferred_element_type=jnp.float32) + bt1_ref[...]
    x = jnp.where(x >= 0, x, NEG * x)
    x = jnp.dot(x, wt2_ref[...], preferred_element_type=jnp.float32) + bt2_ref[...]
    x_ref[...] = x
    g_ref[...] = jnp.dot(x, wg_ref[...], preferred_element_type=jnp.float32) + bg_ref[...]


def _pre_chain(h, p):
    grid = (N // _BLK,)
    full = lambda shape: pl.BlockSpec(shape, lambda i: (0, 0))
    return pl.pallas_call(
        _pre_body,
        grid=grid,
        in_specs=[
            pl.BlockSpec((_BLK, DIN), lambda i: (i, 0)),
            full((DIN, H)), full((1, H)),
            full((H, H)), full((1, H)),
            full((H, H)), full((1, H)),
            full((H, 1)), full((1, 1)),
        ],
        out_specs=[
            pl.BlockSpec((_BLK, H), lambda i: (i, 0)),
            pl.BlockSpec((_BLK, 1), lambda i: (i, 0)),
        ],
        out_shape=[
            jax.ShapeDtypeStruct((N, H), jnp.float32),
            jax.ShapeDtypeStruct((N, 1), jnp.float32),
        ],
    )(h, p['W_in'], p['b_in'][None], p['W_t1'], p['b_t1'][None],
      p['W_t2'], p['b_t2'][None], p['W_gate'], p['b_gate'][None])


def _deg_body(ei_ref, out_src, out_dst, idx_v, ones_v, zbuf, shared):
    c = lax.axis_index("c")
    s = lax.axis_index("s")
    for k in range(EW // LANES):
        ones_v[pl.ds(LANES * k, LANES)] = jnp.ones((LANES,), jnp.float32)

    def zb(i, _):
        zbuf[pl.ds(i * LANES, LANES)] = jnp.zeros((LANES,), jnp.float32)
        return 0
    lax.fori_loop(0, NPT // LANES, zb, 0)
    pltpu.sync_copy(zbuf, shared.at[pl.ds(s * NPT, NPT)])
    plsc.subcore_barrier()

    for cs in range(NC):
        @pl.when(c == cs)
        def _():
            def blk(b, _):
                base = s * RPT + b * KB
                pltpu.sync_copy(ei_ref.at[cs, pl.ds(base, KB)], idx_v)
                for j in range(KB):
                    pltpu.sync_copy(ones_v, shared.at[idx_v.at[j]], add=True)
                return 0
            lax.fori_loop(0, NBLK, blk, 0)
            plsc.subcore_barrier()
            out = out_src if cs == 0 else out_dst
            pltpu.sync_copy(shared.at[pl.ds(s * NPT, NPT)], zbuf)
            pltpu.sync_copy(zbuf, out.at[pl.ds(s * NPT, NPT)])


def _degrees(ei2):
    mesh = plsc.VectorSubcoreMesh(core_axis_name="c", subcore_axis_name="s")
    f = pl.kernel(
        _deg_body,
        out_type=[jax.ShapeDtypeStruct((N_PAD,), jnp.float32),
                  jax.ShapeDtypeStruct((N_PAD,), jnp.float32)],
        mesh=mesh,
        scratch_types=[
            pltpu.VMEM((KB, EW), jnp.int32),
            pltpu.VMEM((EW,), jnp.float32),
            pltpu.VMEM((NPT,), jnp.float32),
            pltpu.VMEM_SHARED((N_PAD,), jnp.float32),
        ],
    )
    return f(ei2)


def _agg_body(xs0, xs1, xs2, xs3, ei_ref, o0, o1, o2, o3,
              sidx, didx, rows_v, zbuf2, bounce, shared2, sem):
    c = lax.axis_index("c")
    s = lax.axis_index("s")
    xs_refs = (xs0, xs1, xs2, xs3)
    out_refs = (o0, o1, o2, o3)

    def zb(i, _):
        zbuf2[i, :] = jnp.zeros((LANES,), jnp.float32)
        return 0
    lax.fori_loop(0, CZ, zb, 0)

    for cs in range(NC):
        @pl.when(c == cs)
        def _():
            for cc in range(2):
                fch = cs * 2 + cc
                for k in range(NCZ):
                    pltpu.sync_copy(
                        zbuf2, shared2.at[pl.ds(s * NPT + k * CZ, CZ)])
                plsc.subcore_barrier()

                def blk(b, _):
                    base = s * RPT + b * KB
                    pltpu.sync_copy(ei_ref.at[0, pl.ds(base, KB)], sidx)
                    pltpu.sync_copy(ei_ref.at[1, pl.ds(base, KB)], didx)
                    cps = [pltpu.async_copy(xs_refs[fch].at[sidx.at[j]],
                                            rows_v.at[j], sem)
                           for j in range(KB)]
                    for cp in cps:
                        cp.wait()
                    for j in range(KB):
                        pltpu.sync_copy(rows_v.at[j], shared2.at[didx.at[j]],
                                        add=True)
                    return 0
                lax.fori_loop(0, NBLK, blk, 0)
                plsc.subcore_barrier()
                for k in range(NCZ):
                    pltpu.sync_copy(
                        shared2.at[pl.ds(s * NPT + k * CZ, CZ)], bounce)
                    pltpu.sync_copy(
                        bounce, out_refs[fch].at[pl.ds(s * NPT + k * CZ, CZ)])
                plsc.subcore_barrier()


def _gcn_agg(xs_chunks, ei2):
    mesh = plsc.VectorSubcoreMesh(core_axis_name="c", subcore_axis_name="s")
    f = pl.kernel(
        _agg_body,
        out_type=[jax.ShapeDtypeStruct((N_PAD, LANES), jnp.float32)] * 4,
        mesh=mesh,
        compiler_params=pltpu.CompilerParams(use_tc_tiling_on_sc=False),
        scratch_types=[
            pltpu.VMEM((KB, EW), jnp.int32),
            pltpu.VMEM((KB, EW), jnp.int32),
            pltpu.VMEM((KB, EW, LANES), jnp.float32),
            pltpu.VMEM((CZ, LANES), jnp.float32),
            pltpu.VMEM((CZ, LANES), jnp.float32),
            pltpu.VMEM_SHARED((N_PAD, LANES), jnp.float32),
            pltpu.SemaphoreType.DMA,
        ],
    )
    return f(*xs_chunks, ei2)


def kernel(h, params, edge_index):
    p = params
    ei2 = jnp.concatenate(
        [edge_index, jnp.full((2, E_PAD - E), N, jnp.int32)], axis=1
    ).reshape(2, ROWS, EW)

    cnt_src, cnt_dst = _degrees(ei2)
    deg_out = jnp.maximum(cnt_src[:N], 1.0)
    deg_in = jnp.maximum(cnt_dst[:N], 1.0)

    x, gate = _pre_chain(h, p)

    xs = x * (deg_out ** -0.5)[:, None]
    xs_pad = jnp.pad(xs, ((0, N_PAD - N), (0, 0)))
    xs_chunks = [xs_pad[:, 16 * f:16 * f + 16] for f in range(4)]
    agg4 = _gcn_agg(xs_chunks, ei2)
    agg = jnp.concatenate([a[:N] for a in agg4], axis=1)
    agg = agg * (deg_in ** -0.5)[:, None]

    gcn_h = _ln(agg @ p['W_gcn'] + p['b_gcn'], p['ln_gcn_g'], p['ln_gcn_b'])
    loc_h = gcn_h - x
    a = jax.nn.softmax(gate, axis=0)
    pool = jnp.sum(a * x, axis=0, keepdims=True)
    glo_h = jnp.broadcast_to(pool, x.shape) - x
    Z = jnp.concatenate([loc_h, glo_h], axis=1)
    hh = Z
    src = edge_index[0]
    dst = edge_index[1]
    for lp in p['layers']:
        hp = jax.nn.relu(hh @ lp['Wp'] + lp['bp'])
        neigh = jax.ops.segment_max(hp[src], dst, num_segments=N)
        neigh = jnp.where(jnp.isfinite(neigh), neigh, 0.0)
        out = hh @ lp['Ws'] + neigh @ lp['Wn'] + lp['bs']
        hh = jax.nn.relu(_ln(out, lp['ln_g'], lp['ln_b']))
    score = hh @ p['W_out'] + p['b_out']
    return score, Z


# speedup vs baseline: 2.1346x; 1.4415x over previous
import functools

import jax
import jax.numpy as jnp
from jax import lax
from jax.experimental import pallas as pl
from jax.experimental.pallas import tpu as pltpu
from jax.experimental.pallas import tpu_sc as plsc

N = 100000
E = 1600000
DIN = 17
H = 64
H2 = 128
OUT = 2
NEG = 0.05
EPS = 1e-5

NC, NS, LANES = 2, 16, 16
N_PAD = 100096
NPT = N_PAD // NS
ZR = NPT // 8
EW = 128
ROWS = 12544
E_PAD = ROWS * EW
RPT = ROWS // NS
KB = 4
NBLK = RPT // KB
CZ = 391
NCZ = NPT // CZ

_BLK = 4000


def _ln(x, g, b):
    m = jnp.mean(x, axis=-1, keepdims=True)
    v = jnp.var(x, axis=-1, keepdims=True)
    return (x - m) / jnp.sqrt(v + EPS) * g + b


def _pre_body(h_ref, win_ref, bin_ref, wt1_ref, bt1_ref, wt2_ref, bt2_ref,
              wg_ref, bg_ref, x_ref, g_ref):
    x = jnp.dot(h_ref[...], win_ref[...], preferred_element_type=jnp.float32)
    x = x + bin_ref[...]
    x = jnp.dot(x, wt1_ref[...], preferred_element_type=jnp.float32) + bt1_ref[...]
    x = jnp.where(x >= 0, x, NEG * x)
    x = jnp.dot(x, wt2_ref[...], preferred_element_type=jnp.float32) + bt2_ref[...]
    x_ref[...] = x
    g_ref[...] = jnp.dot(x, wg_ref[...], preferred_element_type=jnp.float32) + bg_ref[...]


def _pre_chain(h, p):
    grid = (N // _BLK,)
    full = lambda shape: pl.BlockSpec(shape, lambda i: (0, 0))
    return pl.pallas_call(
        _pre_body,
        grid=grid,
        in_specs=[
            pl.BlockSpec((_BLK, DIN), lambda i: (i, 0)),
            full((DIN, H)), full((1, H)),
            full((H, H)), full((1, H)),
            full((H, H)), full((1, H)),
            full((H, 1)), full((1, 1)),
        ],
        out_specs=[
            pl.BlockSpec((_BLK, H), lambda i: (i, 0)),
            pl.BlockSpec((_BLK, 1), lambda i: (i, 0)),
        ],
        out_shape=[
            jax.ShapeDtypeStruct((N, H), jnp.float32),
            jax.ShapeDtypeStruct((N, 1), jnp.float32),
        ],
    )(h, p['W_in'], p['b_in'][None], p['W_t1'], p['b_t1'][None],
      p['W_t2'], p['b_t2'][None], p['W_gate'], p['b_gate'][None])


def _deg_body(ei_ref, out_src, out_dst, idx_v, ones_v, zbuf, shared):
    c = lax.axis_index("c")
    s = lax.axis_index("s")
    for k in range(EW // LANES):
        ones_v[pl.ds(LANES * k, LANES)] = jnp.ones((LANES,), jnp.float32)

    def zb(i, _):
        zbuf[pl.ds(i * LANES, LANES)] = jnp.zeros((LANES,), jnp.float32)
        return 0
    lax.fori_loop(0, NPT // LANES, zb, 0)
    pltpu.sync_copy(zbuf, shared.at[pl.ds(s * NPT, NPT)])
    plsc.subcore_barrier()

    for cs in range(NC):
        @pl.when(c == cs)
        def _():
            def blk(b, _):
                base = s * RPT + b * KB
                pltpu.sync_copy(ei_ref.at[cs, pl.ds(base, KB)], idx_v)
                for j in range(KB):
                    pltpu.sync_copy(ones_v, shared.at[idx_v.at[j]], add=True)
                return 0
            lax.fori_loop(0, NBLK, blk, 0)
            plsc.subcore_barrier()
            out = out_src if cs == 0 else out_dst
            pltpu.sync_copy(shared.at[pl.ds(s * NPT, NPT)], zbuf)
            pltpu.sync_copy(zbuf, out.at[pl.ds(s * NPT, NPT)])


def _degrees(ei2):
    mesh = plsc.VectorSubcoreMesh(core_axis_name="c", subcore_axis_name="s")
    f = pl.kernel(
        _deg_body,
        out_type=[jax.ShapeDtypeStruct((N_PAD,), jnp.float32),
                  jax.ShapeDtypeStruct((N_PAD,), jnp.float32)],
        mesh=mesh,
        scratch_types=[
            pltpu.VMEM((KB, EW), jnp.int32),
            pltpu.VMEM((EW,), jnp.float32),
            pltpu.VMEM((NPT,), jnp.float32),
            pltpu.VMEM_SHARED((N_PAD,), jnp.float32),
        ],
    )
    return f(ei2)


def _agg_body(xs0, xs1, xs2, xs3, ei_ref, o0, o1, o2, o3,
              sidx, didx, rows_v, zbuf2, bounce, shared2, sem):
    c = lax.axis_index("c")
    s = lax.axis_index("s")
    xs_refs = (xs0, xs1, xs2, xs3)
    out_refs = (o0, o1, o2, o3)

    def zb(i, _):
        zbuf2[i, :] = jnp.zeros((LANES,), jnp.float32)
        return 0
    lax.fori_loop(0, CZ, zb, 0)

    for cs in range(NC):
        @pl.when(c == cs)
        def _():
            for cc in range(2):
                fch = cs * 2 + cc
                for k in range(NCZ):
                    pltpu.sync_copy(
                        zbuf2, shared2.at[pl.ds(s * NPT + k * CZ, CZ)])
                plsc.subcore_barrier()

                def blk(b, _):
                    base = s * RPT + b * KB
                    pltpu.sync_copy(ei_ref.at[0, pl.ds(base, KB)], sidx)
                    pltpu.sync_copy(ei_ref.at[1, pl.ds(base, KB)], didx)
                    cps = [pltpu.async_copy(xs_refs[fch].at[sidx.at[j]],
                                            rows_v.at[j], sem)
                           for j in range(KB)]
                    for cp in cps:
                        cp.wait()
                    for j in range(KB):
                        pltpu.sync_copy(rows_v.at[j], shared2.at[didx.at[j]],
                                        add=True)
                    return 0
                lax.fori_loop(0, NBLK, blk, 0)
                plsc.subcore_barrier()
                for k in range(NCZ):
                    pltpu.sync_copy(
                        shared2.at[pl.ds(s * NPT + k * CZ, CZ)], bounce)
                    pltpu.sync_copy(
                        bounce, out_refs[fch].at[pl.ds(s * NPT + k * CZ, CZ)])
                plsc.subcore_barrier()


def _gcn_agg(xs_chunks, ei2):
    mesh = plsc.VectorSubcoreMesh(core_axis_name="c", subcore_axis_name="s")
    f = pl.kernel(
        _agg_body,
        out_type=[jax.ShapeDtypeStruct((N_PAD, LANES), jnp.float32)] * 4,
        mesh=mesh,
        compiler_params=pltpu.CompilerParams(use_tc_tiling_on_sc=False),
        scratch_types=[
            pltpu.VMEM((KB, EW), jnp.int32),
            pltpu.VMEM((KB, EW), jnp.int32),
            pltpu.VMEM((KB, EW, LANES), jnp.float32),
            pltpu.VMEM((CZ, LANES), jnp.float32),
            pltpu.VMEM((CZ, LANES), jnp.float32),
            pltpu.VMEM_SHARED((N_PAD, LANES), jnp.float32),
            pltpu.SemaphoreType.DMA,
        ],
    )
    return f(*xs_chunks, ei2)


NB = 196
NBP = 224
BSP = 224
NPAD2 = NB * 512
KB2 = 8
RPT2 = ROWS // (NC * NS)
NBLK2 = RPT2 // KB2
GE = E_PAD + NB * 16

_IOTA = lambda: lax.iota(jnp.int32, LANES)


def _vgather(x, idx):
    return lax.gather(
        x, idx[:, None],
        lax.GatherDimensionNumbers(offset_dims=(), collapsed_slice_dims=(0,),
                                   start_index_map=(0,)),
        (1,), mode=lax.GatherScatterMode.PROMISE_IN_BOUNDS)


def _runs(bvec):
    it = _IOTA()
    bs, perm = plsc.sort_key_val(bvec, it)
    prev = _vgather(bs, jnp.maximum(it - 1, 0))
    nxt = _vgather(bs, jnp.minimum(it + 1, LANES - 1))
    isstart = (it == 0) | (bs != prev)
    isend = (it == LANES - 1) | (bs != nxt)
    ri = it - plsc.cummax(jnp.where(isstart, it, 0))
    return bs, perm, ri, isend


def _b1_body(ei_ref, cnt_out, dbuf, cnt_v):
    c = lax.axis_index("c")
    s = lax.axis_index("s")
    wid = c * NS + s
    for k in range(NBP // LANES):
        cnt_v[pl.ds(k * LANES, LANES)] = jnp.zeros((LANES,), jnp.int32)

    def blk(i, _):
        pltpu.sync_copy(ei_ref.at[1, pl.ds(wid * RPT2 + i * KB2, KB2)], dbuf)
        for j in range(KB2):
            for k in range(EW // LANES):
                b = dbuf[j, pl.ds(k * LANES, LANES)] >> 9
                bs, _p, ri, isend = _runs(b)
                cnts = plsc.load_gather(cnt_v, [bs])
                plsc.store_scatter(cnt_v, [bs], cnts + ri + 1, mask=isend)
        return 0
    lax.fori_loop(0, NBLK2, blk, 0)
    pltpu.sync_copy(cnt_v, cnt_out.at[wid])


def _bucket_count(ei2):
    mesh = plsc.VectorSubcoreMesh(core_axis_name="c", subcore_axis_name="s")
    f = pl.kernel(
        _b1_body,
        out_type=jax.ShapeDtypeStruct((32, NBP), jnp.int32),
        mesh=mesh,
        compiler_params=pltpu.CompilerParams(needs_layout_passes=False),
        scratch_types=[
            pltpu.VMEM((KB2, EW), jnp.int32),
            pltpu.VMEM((NBP,), jnp.int32),
        ],
    )
    return f(ei2)


def _b2_body(ei_ref, offs_ref, g_src, g_dst,
             sbuf, dbuf, posb, svb, dvb, pos_v, sem):
    c = lax.axis_index("c")
    s = lax.axis_index("s")
    wid = c * NS + s
    pltpu.sync_copy(offs_ref.at[wid], pos_v)

    def blk(i, _):
        pltpu.sync_copy(ei_ref.at[0, pl.ds(wid * RPT2 + i * KB2, KB2)], sbuf)
        pltpu.sync_copy(ei_ref.at[1, pl.ds(wid * RPT2 + i * KB2, KB2)], dbuf)
        for j in range(KB2):
            for k in range(EW // LANES):
                sv = sbuf[j, pl.ds(k * LANES, LANES)]
                dv = dbuf[j, pl.ds(k * LANES, LANES)]
                b = dv >> 9
                bs, perm, ri, isend = _runs(b)
                svs = _vgather(sv, perm)
                dvs = _vgather(dv, perm)
                curs = plsc.load_gather(pos_v, [bs])
                pos = curs + ri
                plsc.store_scatter(pos_v, [bs], pos + 1, mask=isend)
                posb[j, pl.ds(k * LANES, LANES)] = pos
                svb[j, pl.ds(k * LANES, LANES)] = svs
                dvb[j, pl.ds(k * LANES, LANES)] = dvs
        cps = [pltpu.async_copy(svb.at[j], g_src.at[posb.at[j]], sem)
               for j in range(KB2)]
        cps += [pltpu.async_copy(dvb.at[j], g_dst.at[posb.at[j]], sem)
                for j in range(KB2)]
        for cp in cps:
            cp.wait()
        return 0
    lax.fori_loop(0, NBLK2, blk, 0)


def _bucket_scatter(ei2, offs):
    mesh = plsc.VectorSubcoreMesh(core_axis_name="c", subcore_axis_name="s")
    f = pl.kernel(
        _b2_body,
        out_type=[jax.ShapeDtypeStruct((GE,), jnp.int32),
                  jax.ShapeDtypeStruct((GE,), jnp.int32)],
        mesh=mesh,
        compiler_params=pltpu.CompilerParams(needs_layout_passes=False),
        scratch_types=[
            pltpu.VMEM((KB2, EW), jnp.int32),
            pltpu.VMEM((KB2, EW), jnp.int32),
            pltpu.VMEM((KB2, EW), jnp.int32),
            pltpu.VMEM((KB2, EW), jnp.int32),
            pltpu.VMEM((KB2, EW), jnp.int32),
            pltpu.VMEM((NBP,), jnp.int32),
            pltpu.SemaphoreType.DMA,
        ],
    )
    return f(ei2, offs)


def _smax_body(gs_ref, gd_ref, hp_ref, lo_ref, ln_ref, out_ref,
               sb, sb16, db, lov, lnv, rows_v, acc, sem):
    c = lax.axis_index("c")
    s = lax.axis_index("s")
    wid = c * NS + s
    pltpu.sync_copy(lo_ref.at[wid], lov)
    pltpu.sync_copy(ln_ref.at[wid], lnv)
    lo_all = lov[pl.ds(0, LANES)]
    ln_all = lnv[pl.ds(0, LANES)]

    for k in range(7):
        b = k * 32 + wid

        @pl.when(b < NB)
        def _():
            lo = lo_all[k]
            ln = ln_all[k]
            base = b * 512

            def za(i, _):
                for f8 in range(8):
                    acc[i, pl.ds(f8 * LANES, LANES)] = jnp.zeros(
                        (LANES,), jnp.float32)
                return 0
            lax.fori_loop(0, 520, za, 0)

            def sub(off, cnt):
                def grp(g, _):
                    dv16 = db[pl.ds(g * 8, 16)]
                    for j in range(8):
                        e = g * 8 + j
                        dl = dv16[j] - base
                        dl = jnp.where(
                            (off + e < ln) & (dl >= 0) & (dl < 512), dl, 512)
                        for f8 in range(8):
                            acc[dl, pl.ds(f8 * LANES, LANES)] = jnp.maximum(
                                acc[dl, pl.ds(f8 * LANES, LANES)],
                                rows_v[e, pl.ds(f8 * LANES, LANES)])
                    return 0
                lax.fori_loop(0, cnt >> 3, grp, 0)

            n128 = ln >> 7

            def big(i, _):
                p = pl.multiple_of(lo + i * EW, 16)
                pltpu.sync_copy(gs_ref.at[pl.ds(p, EW)], sb)
                pltpu.sync_copy(gd_ref.at[pl.ds(p, EW)], db.at[pl.ds(0, EW)])
                for q in range(EW // LANES):
                    v = sb[pl.ds(q * LANES, LANES)]
                    sb[pl.ds(q * LANES, LANES)] = jnp.clip(v, 0, N)
                pltpu.async_copy(hp_ref.at[sb], rows_v, sem).wait()
                sub(i * EW, EW)
                return 0
            lax.fori_loop(0, n128, big, 0)

            def rem(i, _):
                p = pl.multiple_of(lo + n128 * EW + i * 16, 16)
                pltpu.sync_copy(gs_ref.at[pl.ds(p, 16)], sb16)
                pltpu.sync_copy(gd_ref.at[pl.ds(p, 16)], db.at[pl.ds(0, 16)])
                v = sb16[pl.ds(0, LANES)]
                sb16[pl.ds(0, LANES)] = jnp.clip(v, 0, N)
                pltpu.async_copy(hp_ref.at[sb16],
                                 rows_v.at[pl.ds(0, 16)], sem).wait()
                sub(n128 * EW + i * 16, 16)
                return 0
            lax.fori_loop(0, ((ln + 15) >> 4) - (n128 << 3), rem, 0)

            pltpu.sync_copy(acc.at[pl.ds(0, 512)],
                            out_ref.at[pl.ds(base, 512)])


def _segmax(gs, gd, hp_pad, loT, lnT):
    mesh = plsc.VectorSubcoreMesh(core_axis_name="c", subcore_axis_name="s")
    f = pl.kernel(
        _smax_body,
        out_type=jax.ShapeDtypeStruct((NPAD2, H2), jnp.float32),
        mesh=mesh,
        compiler_params=pltpu.CompilerParams(needs_layout_passes=False),
        scratch_types=[
            pltpu.VMEM((EW,), jnp.int32),
            pltpu.VMEM((16,), jnp.int32),
            pltpu.VMEM((EW + 16,), jnp.int32),
            pltpu.VMEM((LANES,), jnp.int32),
            pltpu.VMEM((LANES,), jnp.int32),
            pltpu.VMEM((EW, H2), jnp.float32),
            pltpu.VMEM((521, H2), jnp.float32),
            pltpu.SemaphoreType.DMA,
        ],
    )
    return f(gs, gd, hp_pad, loT, lnT)


def _group_edges(ei2):
    cnt = _bucket_count(ei2)[:, :NB]
    tot = jnp.sum(cnt, axis=0)
    tot16 = (tot + 15) & ~15
    bst = jnp.concatenate([jnp.zeros((1,), jnp.int32),
                           jnp.cumsum(tot16, dtype=jnp.int32)])
    excl = jnp.cumsum(cnt, axis=0, dtype=jnp.int32) - cnt
    offs = bst[None, :NB] + excl
    offs = jnp.pad(offs, ((0, 0), (0, NBP - NB)))
    gs, gd = _bucket_scatter(ei2, offs)
    bstp = jnp.pad(bst[:NB], (0, 224 - NB)).reshape(7, 32).T
    totp = jnp.pad(tot, (0, 224 - NB)).reshape(7, 32).T
    loT = jnp.pad(bstp, ((0, 0), (0, LANES - 7)))
    lnT = jnp.pad(totp, ((0, 0), (0, LANES - 7)))
    return gs, gd, loT, lnT


def kernel(h, params, edge_index):
    p = params
    ei2 = jnp.concatenate(
        [edge_index, jnp.full((2, E_PAD - E), N, jnp.int32)], axis=1
    ).reshape(2, ROWS, EW)

    cnt_src, cnt_dst = _degrees(ei2)
    deg_out = jnp.maximum(cnt_src[:N], 1.0)
    deg_in = jnp.maximum(cnt_dst[:N], 1.0)

    x, gate = _pre_chain(h, p)

    xs = x * (deg_out ** -0.5)[:, None]
    xs_pad = jnp.pad(xs, ((0, N_PAD - N), (0, 0)))
    xs_chunks = [xs_pad[:, 16 * f:16 * f + 16] for f in range(4)]
    agg4 = _gcn_agg(xs_chunks, ei2)
    agg = jnp.concatenate([a[:N] for a in agg4], axis=1)
    agg = agg * (deg_in ** -0.5)[:, None]

    gcn_h = _ln(agg @ p['W_gcn'] + p['b_gcn'], p['ln_gcn_g'], p['ln_gcn_b'])
    loc_h = gcn_h - x
    a = jax.nn.softmax(gate, axis=0)
    pool = jnp.sum(a * x, axis=0, keepdims=True)
    glo_h = jnp.broadcast_to(pool, x.shape) - x
    Z = jnp.concatenate([loc_h, glo_h], axis=1)
    hh = Z
    gs, gd, loT, lnT = _group_edges(ei2)
    for lp in p['layers']:
        hp = jax.nn.relu(hh @ lp['Wp'] + lp['bp'])
        hp_pad = jnp.pad(hp, ((0, NPAD2 - N), (0, 0)))
        neigh = _segmax(gs, gd, hp_pad, loT, lnT)[:N]
        out = hh @ lp['Ws'] + neigh @ lp['Wn'] + lp['bs']
        hh = jax.nn.relu(_ln(out, lp['ln_g'], lp['ln_b']))
    score = hh @ p['W_out'] + p['b_out']
    return score, Z


# speedup vs baseline: 2.8534x; 1.3368x over previous
import functools

import jax
import jax.numpy as jnp
from jax import lax
from jax.experimental import pallas as pl
from jax.experimental.pallas import tpu as pltpu
from jax.experimental.pallas import tpu_sc as plsc

N = 100000
E = 1600000
DIN = 17
H = 64
H2 = 128
OUT = 2
NEG = 0.05
EPS = 1e-5

NC, NS, LANES = 2, 16, 16
N_PAD = 100096
NPT = N_PAD // NS
ZR = NPT // 8
EW = 128
ROWS = 12544
E_PAD = ROWS * EW
RPT = ROWS // NS
KB = 4
NBLK = RPT // KB
CZ = 391
NCZ = NPT // CZ

_BLK = 4000


def _ln(x, g, b):
    m = jnp.mean(x, axis=-1, keepdims=True)
    v = jnp.var(x, axis=-1, keepdims=True)
    return (x - m) / jnp.sqrt(v + EPS) * g + b


def _pre_body(h_ref, win_ref, bin_ref, wt1_ref, bt1_ref, wt2_ref, bt2_ref,
              wg_ref, bg_ref, x_ref, g_ref):
    x = jnp.dot(h_ref[...], win_ref[...], preferred_element_type=jnp.float32)
    x = x + bin_ref[...]
    x = jnp.dot(x, wt1_ref[...], preferred_element_type=jnp.float32) + bt1_ref[...]
    x = jnp.where(x >= 0, x, NEG * x)
    x = jnp.dot(x, wt2_ref[...], preferred_element_type=jnp.float32) + bt2_ref[...]
    x_ref[...] = x
    g_ref[...] = jnp.dot(x, wg_ref[...], preferred_element_type=jnp.float32) + bg_ref[...]


def _pre_chain(h, p):
    grid = (N // _BLK,)
    full = lambda shape: pl.BlockSpec(shape, lambda i: (0, 0))
    return pl.pallas_call(
        _pre_body,
        grid=grid,
        in_specs=[
            pl.BlockSpec((_BLK, DIN), lambda i: (i, 0)),
            full((DIN, H)), full((1, H)),
            full((H, H)), full((1, H)),
            full((H, H)), full((1, H)),
            full((H, 1)), full((1, 1)),
        ],
        out_specs=[
            pl.BlockSpec((_BLK, H), lambda i: (i, 0)),
            pl.BlockSpec((_BLK, 1), lambda i: (i, 0)),
        ],
        out_shape=[
            jax.ShapeDtypeStruct((N, H), jnp.float32),
            jax.ShapeDtypeStruct((N, 1), jnp.float32),
        ],
    )(h, p['W_in'], p['b_in'][None], p['W_t1'], p['b_t1'][None],
      p['W_t2'], p['b_t2'][None], p['W_gate'], p['b_gate'][None])


def _deg_body(ei_ref, out_src, out_dst, idx_v, ones_v, zbuf, shared):
    c = lax.axis_index("c")
    s = lax.axis_index("s")
    for k in range(EW // LANES):
        ones_v[pl.ds(LANES * k, LANES)] = jnp.ones((LANES,), jnp.float32)

    def zb(i, _):
        zbuf[pl.ds(i * LANES, LANES)] = jnp.zeros((LANES,), jnp.float32)
        return 0
    lax.fori_loop(0, NPT // LANES, zb, 0)
    pltpu.sync_copy(zbuf, shared.at[pl.ds(s * NPT, NPT)])
    plsc.subcore_barrier()

    for cs in range(NC):
        @pl.when(c == cs)
        def _():
            def blk(b, _):
                base = s * RPT + b * KB
                pltpu.sync_copy(ei_ref.at[cs, pl.ds(base, KB)], idx_v)
                for j in range(KB):
                    pltpu.sync_copy(ones_v, shared.at[idx_v.at[j]], add=True)
                return 0
            lax.fori_loop(0, NBLK, blk, 0)
            plsc.subcore_barrier()
            out = out_src if cs == 0 else out_dst
            pltpu.sync_copy(shared.at[pl.ds(s * NPT, NPT)], zbuf)
            pltpu.sync_copy(zbuf, out.at[pl.ds(s * NPT, NPT)])


def _degrees(ei2):
    mesh = plsc.VectorSubcoreMesh(core_axis_name="c", subcore_axis_name="s")
    f = pl.kernel(
        _deg_body,
        out_type=[jax.ShapeDtypeStruct((N_PAD,), jnp.float32),
                  jax.ShapeDtypeStruct((N_PAD,), jnp.float32)],
        mesh=mesh,
        scratch_types=[
            pltpu.VMEM((KB, EW), jnp.int32),
            pltpu.VMEM((EW,), jnp.float32),
            pltpu.VMEM((NPT,), jnp.float32),
            pltpu.VMEM_SHARED((N_PAD,), jnp.float32),
        ],
    )
    return f(ei2)


def _agg_body(xs0, xs1, xs2, xs3, ei_ref, o0, o1, o2, o3,
              sidx, didx, rows_v, zbuf2, bounce, shared2, sem):
    c = lax.axis_index("c")
    s = lax.axis_index("s")
    xs_refs = (xs0, xs1, xs2, xs3)
    out_refs = (o0, o1, o2, o3)

    def zb(i, _):
        zbuf2[i, :] = jnp.zeros((LANES,), jnp.float32)
        return 0
    lax.fori_loop(0, CZ, zb, 0)

    for cs in range(NC):
        @pl.when(c == cs)
        def _():
            for cc in range(2):
                fch = cs * 2 + cc
                for k in range(NCZ):
                    pltpu.sync_copy(
                        zbuf2, shared2.at[pl.ds(s * NPT + k * CZ, CZ)])
                plsc.subcore_barrier()

                def blk(b, _):
                    base = s * RPT + b * KB
                    pltpu.sync_copy(ei_ref.at[0, pl.ds(base, KB)], sidx)
                    pltpu.sync_copy(ei_ref.at[1, pl.ds(base, KB)], didx)
                    cps = [pltpu.async_copy(xs_refs[fch].at[sidx.at[j]],
                                            rows_v.at[j], sem)
                           for j in range(KB)]
                    for cp in cps:
                        cp.wait()
                    for j in range(KB):
                        pltpu.sync_copy(rows_v.at[j], shared2.at[didx.at[j]],
                                        add=True)
                    return 0
                lax.fori_loop(0, NBLK, blk, 0)
                plsc.subcore_barrier()
                for k in range(NCZ):
                    pltpu.sync_copy(
                        shared2.at[pl.ds(s * NPT + k * CZ, CZ)], bounce)
                    pltpu.sync_copy(
                        bounce, out_refs[fch].at[pl.ds(s * NPT + k * CZ, CZ)])
                plsc.subcore_barrier()


def _gcn_agg(xs_chunks, ei2):
    mesh = plsc.VectorSubcoreMesh(core_axis_name="c", subcore_axis_name="s")
    f = pl.kernel(
        _agg_body,
        out_type=[jax.ShapeDtypeStruct((N_PAD, LANES), jnp.float32)] * 4,
        mesh=mesh,
        compiler_params=pltpu.CompilerParams(use_tc_tiling_on_sc=False),
        scratch_types=[
            pltpu.VMEM((KB, EW), jnp.int32),
            pltpu.VMEM((KB, EW), jnp.int32),
            pltpu.VMEM((KB, EW, LANES), jnp.float32),
            pltpu.VMEM((CZ, LANES), jnp.float32),
            pltpu.VMEM((CZ, LANES), jnp.float32),
            pltpu.VMEM_SHARED((N_PAD, LANES), jnp.float32),
            pltpu.SemaphoreType.DMA,
        ],
    )
    return f(*xs_chunks, ei2)


NB = 196
NBP = 224
BSP = 224
NPAD2 = NB * 512
KB2 = 8
RPT2 = ROWS // (NC * NS)
NBLK2 = RPT2 // KB2
GE = E_PAD + NB * 16

_IOTA = lambda: lax.iota(jnp.int32, LANES)


def _vgather(x, idx):
    return lax.gather(
        x, idx[:, None],
        lax.GatherDimensionNumbers(offset_dims=(), collapsed_slice_dims=(0,),
                                   start_index_map=(0,)),
        (1,), mode=lax.GatherScatterMode.PROMISE_IN_BOUNDS)


def _runs(bvec):
    it = _IOTA()
    bs, perm = plsc.sort_key_val(bvec, it)
    prev = _vgather(bs, jnp.maximum(it - 1, 0))
    nxt = _vgather(bs, jnp.minimum(it + 1, LANES - 1))
    isstart = (it == 0) | (bs != prev)
    isend = (it == LANES - 1) | (bs != nxt)
    ri = it - plsc.cummax(jnp.where(isstart, it, 0))
    return bs, perm, ri, isend


def _b1_body(ei_ref, cnt_out, dbuf, cnt_v):
    c = lax.axis_index("c")
    s = lax.axis_index("s")
    wid = c * NS + s
    for k in range(NBP // LANES):
        cnt_v[pl.ds(k * LANES, LANES)] = jnp.zeros((LANES,), jnp.int32)

    def blk(i, _):
        pltpu.sync_copy(ei_ref.at[1, pl.ds(wid * RPT2 + i * KB2, KB2)], dbuf)
        for j in range(KB2):
            for k in range(EW // LANES):
                b = dbuf[j, pl.ds(k * LANES, LANES)] >> 9
                bs, _p, ri, isend = _runs(b)
                cnts = plsc.load_gather(cnt_v, [bs])
                plsc.store_scatter(cnt_v, [bs], cnts + ri + 1, mask=isend)
        return 0
    lax.fori_loop(0, NBLK2, blk, 0)
    pltpu.sync_copy(cnt_v, cnt_out.at[wid])


def _bucket_count(ei2):
    mesh = plsc.VectorSubcoreMesh(core_axis_name="c", subcore_axis_name="s")
    f = pl.kernel(
        _b1_body,
        out_type=jax.ShapeDtypeStruct((32, NBP), jnp.int32),
        mesh=mesh,
        compiler_params=pltpu.CompilerParams(needs_layout_passes=False),
        scratch_types=[
            pltpu.VMEM((KB2, EW), jnp.int32),
            pltpu.VMEM((NBP,), jnp.int32),
        ],
    )
    return f(ei2)


def _b2_body(ei_ref, offs_ref, g_pk,
             sbuf, dbuf, posb, pvb, pos_v, sem):
    c = lax.axis_index("c")
    s = lax.axis_index("s")
    wid = c * NS + s
    pltpu.sync_copy(offs_ref.at[wid], pos_v)

    def blk(i, _):
        pltpu.sync_copy(ei_ref.at[0, pl.ds(wid * RPT2 + i * KB2, KB2)], sbuf)
        pltpu.sync_copy(ei_ref.at[1, pl.ds(wid * RPT2 + i * KB2, KB2)], dbuf)
        for j in range(KB2):
            for k in range(EW // LANES):
                sv = sbuf[j, pl.ds(k * LANES, LANES)]
                dv = dbuf[j, pl.ds(k * LANES, LANES)]
                b = dv >> 9
                pv = (sv << 9) | (dv & 511)
                it = _IOTA()
                bs, pvs = plsc.sort_key_val(b, pv)
                prev = _vgather(bs, jnp.maximum(it - 1, 0))
                nxt = _vgather(bs, jnp.minimum(it + 1, LANES - 1))
                isstart = (it == 0) | (bs != prev)
                isend = (it == LANES - 1) | (bs != nxt)
                ri = it - plsc.cummax(jnp.where(isstart, it, 0))
                curs = plsc.load_gather(pos_v, [bs])
                pos = curs + ri
                plsc.store_scatter(pos_v, [bs], pos + 1, mask=isend)
                posb[j, pl.ds(k * LANES, LANES)] = pos
                pvb[j, pl.ds(k * LANES, LANES)] = pvs
        cps = [pltpu.async_copy(pvb.at[j], g_pk.at[posb.at[j]], sem)
               for j in range(KB2)]
        for cp in cps:
            cp.wait()
        return 0
    lax.fori_loop(0, NBLK2, blk, 0)


def _bucket_scatter(ei2, offs):
    mesh = plsc.VectorSubcoreMesh(core_axis_name="c", subcore_axis_name="s")
    f = pl.kernel(
        _b2_body,
        out_type=jax.ShapeDtypeStruct((GE,), jnp.int32),
        mesh=mesh,
        compiler_params=pltpu.CompilerParams(needs_layout_passes=False),
        scratch_types=[
            pltpu.VMEM((KB2, EW), jnp.int32),
            pltpu.VMEM((KB2, EW), jnp.int32),
            pltpu.VMEM((KB2, EW), jnp.int32),
            pltpu.VMEM((KB2, EW), jnp.int32),
            pltpu.VMEM((NBP,), jnp.int32),
            pltpu.SemaphoreType.DMA,
        ],
    )
    return f(ei2, offs)


def _smax_body(gpk_ref, hp_ref, lo_ref, ln_ref, out_ref,
               sb_a, sb_b, pb_a, pb_b, lov, lnv, rows_a, rows_b, acc,
               sem_a, sem_b):
    c = lax.axis_index("c")
    s = lax.axis_index("s")
    wid = c * NS + s
    pltpu.sync_copy(lo_ref.at[wid], lov)
    pltpu.sync_copy(ln_ref.at[wid], lnv)
    lo_all = lov[pl.ds(0, LANES)]
    ln_all = lnv[pl.ds(0, LANES)]

    for k in range(7):
        b = k * 32 + wid

        @pl.when(b < NB)
        def _():
            lo = lo_all[k]
            ln = ln_all[k]
            base = b * 512

            def za(i, _):
                for f8 in range(8):
                    acc[i, pl.ds(f8 * LANES, LANES)] = jnp.zeros(
                        (LANES,), jnp.float32)
                return 0
            lax.fori_loop(0, 520, za, 0)

            n128 = ln >> 7

            def start(i, sb, pb, rows, sem):
                p = pl.multiple_of(lo + i * EW, 16)
                pltpu.sync_copy(gpk_ref.at[pl.ds(p, EW)], pb.at[pl.ds(0, EW)])
                for q in range(EW // LANES):
                    v = pb[pl.ds(q * LANES, LANES)]
                    sb[pl.ds(q * LANES, LANES)] = jnp.clip(v >> 9, 0, N)
                pltpu.async_copy(hp_ref.at[sb], rows, sem)

            def drain(rows, sem):
                pltpu.make_async_copy(
                    hp_ref.at[pl.ds(0, EW)], rows, sem).wait()

            def sub(off, cnt, pb, rows):
                def grp(g, _):
                    dv16 = pb[pl.ds(g * 8, 16)]
                    for j in range(8):
                        e = g * 8 + j
                        dl = dv16[j] & 511
                        dl = jnp.where(off + e < ln, dl, 512)
                        for f8 in range(8):
                            acc[dl, pl.ds(f8 * LANES, LANES)] = jnp.maximum(
                                acc[dl, pl.ds(f8 * LANES, LANES)],
                                rows[e, pl.ds(f8 * LANES, LANES)])
                    return 0
                lax.fori_loop(0, cnt >> 3, grp, 0)

            @pl.when(n128 > 0)
            def _():
                start(0, sb_a, pb_a, rows_a, sem_a)

            def big(i, _):
                even = (i & 1) == 0

                @pl.when(even)
                def _():
                    @pl.when(i + 1 < n128)
                    def _():
                        start(i + 1, sb_b, pb_b, rows_b, sem_b)
                    drain(rows_a, sem_a)
                    sub(i * EW, EW, pb_a, rows_a)

                @pl.when(jnp.logical_not(even))
                def _():
                    @pl.when(i + 1 < n128)
                    def _():
                        start(i + 1, sb_a, pb_a, rows_a, sem_a)
                    drain(rows_b, sem_b)
                    sub(i * EW, EW, pb_b, rows_b)
                return 0
            lax.fori_loop(0, n128, big, 0)

            def rem(i, _):
                p = pl.multiple_of(lo + n128 * EW + i * 16, 16)
                pltpu.sync_copy(gpk_ref.at[pl.ds(p, 16)], pb_a.at[pl.ds(0, 16)])
                v = pb_a[pl.ds(0, LANES)]
                sb_a[pl.ds(0, LANES)] = jnp.clip(v >> 9, 0, N)
                pltpu.async_copy(hp_ref.at[sb_a.at[pl.ds(0, 16)]],
                                 rows_a.at[pl.ds(0, 16)], sem_a).wait()
                sub(n128 * EW + i * 16, 16, pb_a, rows_a)
                return 0
            lax.fori_loop(0, ((ln + 15) >> 4) - (n128 << 3), rem, 0)

            pltpu.sync_copy(acc.at[pl.ds(0, 512)],
                            out_ref.at[pl.ds(base, 512)])


def _segmax(gpk, hp_pad, loT, lnT):
    mesh = plsc.VectorSubcoreMesh(core_axis_name="c", subcore_axis_name="s")
    f = pl.kernel(
        _smax_body,
        out_type=jax.ShapeDtypeStruct((NPAD2, H2), jnp.float32),
        mesh=mesh,
        compiler_params=pltpu.CompilerParams(needs_layout_passes=False),
        scratch_types=[
            pltpu.VMEM((EW,), jnp.int32),
            pltpu.VMEM((EW,), jnp.int32),
            pltpu.VMEM((EW + 16,), jnp.int32),
            pltpu.VMEM((EW + 16,), jnp.int32),
            pltpu.VMEM((LANES,), jnp.int32),
            pltpu.VMEM((LANES,), jnp.int32),
            pltpu.VMEM((EW, H2), jnp.float32),
            pltpu.VMEM((EW, H2), jnp.float32),
            pltpu.VMEM((521, H2), jnp.float32),
            pltpu.SemaphoreType.DMA,
            pltpu.SemaphoreType.DMA,
        ],
    )
    return f(gpk, hp_pad, loT, lnT)


def _group_edges(ei2):
    cnt = _bucket_count(ei2)[:, :NB]
    tot = jnp.sum(cnt, axis=0)
    tot16 = (tot + 15) & ~15
    bst = jnp.concatenate([jnp.zeros((1,), jnp.int32),
                           jnp.cumsum(tot16, dtype=jnp.int32)])
    excl = jnp.cumsum(cnt, axis=0, dtype=jnp.int32) - cnt
    offs = bst[None, :NB] + excl
    offs = jnp.pad(offs, ((0, 0), (0, NBP - NB)))
    gpk = _bucket_scatter(ei2, offs)
    bstp = jnp.pad(bst[:NB], (0, 224 - NB)).reshape(7, 32).T
    totp = jnp.pad(tot, (0, 224 - NB)).reshape(7, 32).T
    loT = jnp.pad(bstp, ((0, 0), (0, LANES - 7)))
    lnT = jnp.pad(totp, ((0, 0), (0, LANES - 7)))
    return gpk, loT, lnT


def kernel(h, params, edge_index):
    p = params
    ei2 = jnp.concatenate(
        [edge_index, jnp.full((2, E_PAD - E), N, jnp.int32)], axis=1
    ).reshape(2, ROWS, EW)

    cnt_src, cnt_dst = _degrees(ei2)
    deg_out = jnp.maximum(cnt_src[:N], 1.0)
    deg_in = jnp.maximum(cnt_dst[:N], 1.0)

    x, gate = _pre_chain(h, p)

    xs = x * (deg_out ** -0.5)[:, None]
    xs_pad = jnp.pad(xs, ((0, N_PAD - N), (0, 0)))
    xs_chunks = [xs_pad[:, 16 * f:16 * f + 16] for f in range(4)]
    agg4 = _gcn_agg(xs_chunks, ei2)
    agg = jnp.concatenate([a[:N] for a in agg4], axis=1)
    agg = agg * (deg_in ** -0.5)[:, None]

    gcn_h = _ln(agg @ p['W_gcn'] + p['b_gcn'], p['ln_gcn_g'], p['ln_gcn_b'])
    loc_h = gcn_h - x
    a = jax.nn.softmax(gate, axis=0)
    pool = jnp.sum(a * x, axis=0, keepdims=True)
    glo_h = jnp.broadcast_to(pool, x.shape) - x
    Z = jnp.concatenate([loc_h, glo_h], axis=1)
    hh = Z
    gpk, loT, lnT = _group_edges(ei2)
    for lp in p['layers']:
        hp = jax.nn.relu(hh @ lp['Wp'] + lp['bp'])
        hp_pad = jnp.pad(hp, ((0, NPAD2 - N), (0, 0)))
        neigh = _segmax(gpk, hp_pad, loT, lnT)[:N]
        out = hh @ lp['Ws'] + neigh @ lp['Wn'] + lp['bs']
        hh = jax.nn.relu(_ln(out, lp['ln_g'], lp['ln_b']))
    score = hh @ p['W_out'] + p['b_out']
    return score, Z


# speedup vs baseline: 2.9108x; 1.0201x over previous
import functools

import jax
import jax.numpy as jnp
from jax import lax
from jax.experimental import pallas as pl
from jax.experimental.pallas import tpu as pltpu
from jax.experimental.pallas import tpu_sc as plsc

N = 100000
E = 1600000
DIN = 17
H = 64
H2 = 128
OUT = 2
NEG = 0.05
EPS = 1e-5

NC, NS, LANES = 2, 16, 16
N_PAD = 100096
NPT = N_PAD // NS
ZR = NPT // 8
EW = 128
ROWS = 12544
E_PAD = ROWS * EW
RPT = ROWS // NS
KB = 4
NBLK = RPT // KB
CZ = 391
NCZ = NPT // CZ

_BLK = 4000


def _ln(x, g, b):
    m = jnp.mean(x, axis=-1, keepdims=True)
    v = jnp.var(x, axis=-1, keepdims=True)
    return (x - m) / jnp.sqrt(v + EPS) * g + b


def _pre_body(h_ref, win_ref, bin_ref, wt1_ref, bt1_ref, wt2_ref, bt2_ref,
              wg_ref, bg_ref, x_ref, g_ref):
    x = jnp.dot(h_ref[...], win_ref[...], preferred_element_type=jnp.float32)
    x = x + bin_ref[...]
    x = jnp.dot(x, wt1_ref[...], preferred_element_type=jnp.float32) + bt1_ref[...]
    x = jnp.where(x >= 0, x, NEG * x)
    x = jnp.dot(x, wt2_ref[...], preferred_element_type=jnp.float32) + bt2_ref[...]
    x_ref[...] = x
    g_ref[...] = jnp.dot(x, wg_ref[...], preferred_element_type=jnp.float32) + bg_ref[...]


def _pre_chain(h, p):
    grid = (N // _BLK,)
    full = lambda shape: pl.BlockSpec(shape, lambda i: (0, 0))
    return pl.pallas_call(
        _pre_body,
        grid=grid,
        in_specs=[
            pl.BlockSpec((_BLK, DIN), lambda i: (i, 0)),
            full((DIN, H)), full((1, H)),
            full((H, H)), full((1, H)),
            full((H, H)), full((1, H)),
            full((H, 1)), full((1, 1)),
        ],
        out_specs=[
            pl.BlockSpec((_BLK, H), lambda i: (i, 0)),
            pl.BlockSpec((_BLK, 1), lambda i: (i, 0)),
        ],
        out_shape=[
            jax.ShapeDtypeStruct((N, H), jnp.float32),
            jax.ShapeDtypeStruct((N, 1), jnp.float32),
        ],
    )(h, p['W_in'], p['b_in'][None], p['W_t1'], p['b_t1'][None],
      p['W_t2'], p['b_t2'][None], p['W_gate'], p['b_gate'][None])


def _deg_body(ei_ref, out_src, out_dst, idx_v, ones_v, zbuf, shared):
    c = lax.axis_index("c")
    s = lax.axis_index("s")
    for k in range(EW // LANES):
        ones_v[pl.ds(LANES * k, LANES)] = jnp.ones((LANES,), jnp.float32)

    def zb(i, _):
        zbuf[pl.ds(i * LANES, LANES)] = jnp.zeros((LANES,), jnp.float32)
        return 0
    lax.fori_loop(0, NPT // LANES, zb, 0)
    pltpu.sync_copy(zbuf, shared.at[pl.ds(s * NPT, NPT)])
    plsc.subcore_barrier()

    for cs in range(NC):
        @pl.when(c == cs)
        def _():
            def blk(b, _):
                base = s * RPT + b * KB
                pltpu.sync_copy(ei_ref.at[cs, pl.ds(base, KB)], idx_v)
                for j in range(KB):
                    pltpu.sync_copy(ones_v, shared.at[idx_v.at[j]], add=True)
                return 0
            lax.fori_loop(0, NBLK, blk, 0)
            plsc.subcore_barrier()
            out = out_src if cs == 0 else out_dst
            pltpu.sync_copy(shared.at[pl.ds(s * NPT, NPT)], zbuf)
            pltpu.sync_copy(zbuf, out.at[pl.ds(s * NPT, NPT)])


def _degrees(ei2):
    mesh = plsc.VectorSubcoreMesh(core_axis_name="c", subcore_axis_name="s")
    f = pl.kernel(
        _deg_body,
        out_type=[jax.ShapeDtypeStruct((N_PAD,), jnp.float32),
                  jax.ShapeDtypeStruct((N_PAD,), jnp.float32)],
        mesh=mesh,
        scratch_types=[
            pltpu.VMEM((KB, EW), jnp.int32),
            pltpu.VMEM((EW,), jnp.float32),
            pltpu.VMEM((NPT,), jnp.float32),
            pltpu.VMEM_SHARED((N_PAD,), jnp.float32),
        ],
    )
    return f(ei2)


def _agg_body(xs0, xs1, xs2, xs3, ei_ref, o0, o1, o2, o3,
              sidx, didx, rows_v, zbuf2, bounce, shared2, sem):
    c = lax.axis_index("c")
    s = lax.axis_index("s")
    xs_refs = (xs0, xs1, xs2, xs3)
    out_refs = (o0, o1, o2, o3)

    def zb(i, _):
        zbuf2[i, :] = jnp.zeros((LANES,), jnp.float32)
        return 0
    lax.fori_loop(0, CZ, zb, 0)

    for cs in range(NC):
        @pl.when(c == cs)
        def _():
            for cc in range(2):
                fch = cs * 2 + cc
                for k in range(NCZ):
                    pltpu.sync_copy(
                        zbuf2, shared2.at[pl.ds(s * NPT + k * CZ, CZ)])
                plsc.subcore_barrier()

                def blk(b, _):
                    base = s * RPT + b * KB
                    pltpu.sync_copy(ei_ref.at[0, pl.ds(base, KB)], sidx)
                    pltpu.sync_copy(ei_ref.at[1, pl.ds(base, KB)], didx)
                    cps = [pltpu.async_copy(xs_refs[fch].at[sidx.at[j]],
                                            rows_v.at[j], sem)
                           for j in range(KB)]
                    for cp in cps:
                        cp.wait()
                    for j in range(KB):
                        pltpu.sync_copy(rows_v.at[j], shared2.at[didx.at[j]],
                                        add=True)
                    return 0
                lax.fori_loop(0, NBLK, blk, 0)
                plsc.subcore_barrier()
                for k in range(NCZ):
                    pltpu.sync_copy(
                        shared2.at[pl.ds(s * NPT + k * CZ, CZ)], bounce)
                    pltpu.sync_copy(
                        bounce, out_refs[fch].at[pl.ds(s * NPT + k * CZ, CZ)])
                plsc.subcore_barrier()


def _gcn_agg(xs_chunks, ei2):
    mesh = plsc.VectorSubcoreMesh(core_axis_name="c", subcore_axis_name="s")
    f = pl.kernel(
        _agg_body,
        out_type=[jax.ShapeDtypeStruct((N_PAD, LANES), jnp.float32)] * 4,
        mesh=mesh,
        compiler_params=pltpu.CompilerParams(use_tc_tiling_on_sc=False),
        scratch_types=[
            pltpu.VMEM((KB, EW), jnp.int32),
            pltpu.VMEM((KB, EW), jnp.int32),
            pltpu.VMEM((KB, EW, LANES), jnp.float32),
            pltpu.VMEM((CZ, LANES), jnp.float32),
            pltpu.VMEM((CZ, LANES), jnp.float32),
            pltpu.VMEM_SHARED((N_PAD, LANES), jnp.float32),
            pltpu.SemaphoreType.DMA,
        ],
    )
    return f(*xs_chunks, ei2)


NB = 196
NBP = 224
BSP = 224
NPAD2 = NB * 512
KB2 = 8
RPT2 = ROWS // (NC * NS)
NBLK2 = RPT2 // KB2
GE = E_PAD + NB * 16

_IOTA = lambda: lax.iota(jnp.int32, LANES)


def _vgather(x, idx):
    return lax.gather(
        x, idx[:, None],
        lax.GatherDimensionNumbers(offset_dims=(), collapsed_slice_dims=(0,),
                                   start_index_map=(0,)),
        (1,), mode=lax.GatherScatterMode.PROMISE_IN_BOUNDS)


def _runs(bvec):
    it = _IOTA()
    bs, perm = plsc.sort_key_val(bvec, it)
    prev = _vgather(bs, jnp.maximum(it - 1, 0))
    nxt = _vgather(bs, jnp.minimum(it + 1, LANES - 1))
    isstart = (it == 0) | (bs != prev)
    isend = (it == LANES - 1) | (bs != nxt)
    ri = it - plsc.cummax(jnp.where(isstart, it, 0))
    return bs, perm, ri, isend


def _b1_body(ei_ref, cnt_out, dbuf, cnt_v):
    c = lax.axis_index("c")
    s = lax.axis_index("s")
    wid = c * NS + s
    for k in range(NBP // LANES):
        cnt_v[pl.ds(k * LANES, LANES)] = jnp.zeros((LANES,), jnp.int32)

    def blk(i, _):
        pltpu.sync_copy(ei_ref.at[1, pl.ds(wid * RPT2 + i * KB2, KB2)], dbuf)
        for j in range(KB2):
            for k in range(EW // LANES):
                b = dbuf[j, pl.ds(k * LANES, LANES)] >> 9
                bs, _p, ri, isend = _runs(b)
                cnts = plsc.load_gather(cnt_v, [bs])
                plsc.store_scatter(cnt_v, [bs], cnts + ri + 1, mask=isend)
        return 0
    lax.fori_loop(0, NBLK2, blk, 0)
    pltpu.sync_copy(cnt_v, cnt_out.at[wid])


def _bucket_count(ei2):
    mesh = plsc.VectorSubcoreMesh(core_axis_name="c", subcore_axis_name="s")
    f = pl.kernel(
        _b1_body,
        out_type=jax.ShapeDtypeStruct((32, NBP), jnp.int32),
        mesh=mesh,
        compiler_params=pltpu.CompilerParams(needs_layout_passes=False),
        scratch_types=[
            pltpu.VMEM((KB2, EW), jnp.int32),
            pltpu.VMEM((NBP,), jnp.int32),
        ],
    )
    return f(ei2)


def _b2_body(ei_ref, offs_ref, g_pk,
             sbuf, dbuf, posb, pvb, pos_v, sem):
    c = lax.axis_index("c")
    s = lax.axis_index("s")
    wid = c * NS + s
    pltpu.sync_copy(offs_ref.at[wid], pos_v)

    def blk(i, _):
        pltpu.sync_copy(ei_ref.at[0, pl.ds(wid * RPT2 + i * KB2, KB2)], sbuf)
        pltpu.sync_copy(ei_ref.at[1, pl.ds(wid * RPT2 + i * KB2, KB2)], dbuf)
        for j in range(KB2):
            for k in range(EW // LANES):
                sv = sbuf[j, pl.ds(k * LANES, LANES)]
                dv = dbuf[j, pl.ds(k * LANES, LANES)]
                b = dv >> 9
                pv = (sv << 9) | (dv & 511)
                it = _IOTA()
                bs, pvs = plsc.sort_key_val(b, pv)
                prev = _vgather(bs, jnp.maximum(it - 1, 0))
                nxt = _vgather(bs, jnp.minimum(it + 1, LANES - 1))
                isstart = (it == 0) | (bs != prev)
                isend = (it == LANES - 1) | (bs != nxt)
                ri = it - plsc.cummax(jnp.where(isstart, it, 0))
                curs = plsc.load_gather(pos_v, [bs])
                pos = curs + ri
                plsc.store_scatter(pos_v, [bs], pos + 1, mask=isend)
                posb[j, pl.ds(k * LANES, LANES)] = pos
                pvb[j, pl.ds(k * LANES, LANES)] = pvs
        cps = [pltpu.async_copy(pvb.at[j], g_pk.at[posb.at[j]], sem)
               for j in range(KB2)]
        for cp in cps:
            cp.wait()
        return 0
    lax.fori_loop(0, NBLK2, blk, 0)


def _bucket_scatter(ei2, offs):
    mesh = plsc.VectorSubcoreMesh(core_axis_name="c", subcore_axis_name="s")
    f = pl.kernel(
        _b2_body,
        out_type=jax.ShapeDtypeStruct((GE,), jnp.int32),
        mesh=mesh,
        compiler_params=pltpu.CompilerParams(needs_layout_passes=False),
        scratch_types=[
            pltpu.VMEM((KB2, EW), jnp.int32),
            pltpu.VMEM((KB2, EW), jnp.int32),
            pltpu.VMEM((KB2, EW), jnp.int32),
            pltpu.VMEM((KB2, EW), jnp.int32),
            pltpu.VMEM((NBP,), jnp.int32),
            pltpu.SemaphoreType.DMA,
        ],
    )
    return f(ei2, offs)


def _smax_body(gpk_ref, hp_ref, lo_ref, ln_ref, out_ref,
               sb_a, sb_b, pb_a, pb_b, lov, lnv, rows_a, rows_b, acc,
               sem_a, sem_b):
    c = lax.axis_index("c")
    s = lax.axis_index("s")
    wid = c * NS + s
    pltpu.sync_copy(lo_ref.at[wid], lov)
    pltpu.sync_copy(ln_ref.at[wid], lnv)
    lo_all = lov[pl.ds(0, LANES)]
    ln_all = lnv[pl.ds(0, LANES)]

    for k in range(7):
        b = k * 32 + wid

        @pl.when(b < NB)
        def _():
            lo = lo_all[k]
            ln = ln_all[k]
            base = b * 512

            def za(i, _):
                for f8 in range(8):
                    acc[i, pl.ds(f8 * LANES, LANES)] = jnp.zeros(
                        (LANES,), jnp.float32)
                return 0
            lax.fori_loop(0, 520, za, 0)

            n128 = ln >> 7

            def start(i, sb, pb, rows, sem):
                p = pl.multiple_of(lo + i * EW, 16)
                pltpu.sync_copy(gpk_ref.at[pl.ds(p, EW)], pb.at[pl.ds(0, EW)])
                for q in range(EW // LANES):
                    v = pb[pl.ds(q * LANES, LANES)]
                    sb[pl.ds(q * LANES, LANES)] = jnp.clip(v >> 9, 0, N)
                pltpu.async_copy(hp_ref.at[sb], rows, sem)

            def drain(rows, sem):
                pltpu.make_async_copy(
                    hp_ref.at[pl.ds(0, EW)], rows, sem).wait()

            def sub(off, cnt, pb, rows):
                def grp(g, _):
                    dv16 = pb[pl.ds(g * 8, 16)]
                    for j in range(8):
                        e = g * 8 + j
                        dl = dv16[j] & 511
                        dl = jnp.where(off + e < ln, dl, 512)
                        for f8 in range(8):
                            acc[dl, pl.ds(f8 * LANES, LANES)] = jnp.maximum(
                                acc[dl, pl.ds(f8 * LANES, LANES)],
                                rows[e, pl.ds(f8 * LANES, LANES)])
                    return 0
                lax.fori_loop(0, cnt >> 3, grp, 0)

            @pl.when(n128 > 0)
            def _():
                start(0, sb_a, pb_a, rows_a, sem_a)

            def big(i, _):
                even = (i & 1) == 0

                @pl.when(even)
                def _():
                    @pl.when(i + 1 < n128)
                    def _():
                        start(i + 1, sb_b, pb_b, rows_b, sem_b)
                    drain(rows_a, sem_a)
                    sub(i * EW, EW, pb_a, rows_a)

                @pl.when(jnp.logical_not(even))
                def _():
                    @pl.when(i + 1 < n128)
                    def _():
                        start(i + 1, sb_a, pb_a, rows_a, sem_a)
                    drain(rows_b, sem_b)
                    sub(i * EW, EW, pb_b, rows_b)
                return 0
            lax.fori_loop(0, n128, big, 0)

            def rem(i, _):
                p = pl.multiple_of(lo + n128 * EW + i * 16, 16)
                pltpu.sync_copy(gpk_ref.at[pl.ds(p, 16)], pb_a.at[pl.ds(0, 16)])
                v = pb_a[pl.ds(0, LANES)]
                sb_a[pl.ds(0, LANES)] = jnp.clip(v >> 9, 0, N)
                pltpu.async_copy(hp_ref.at[sb_a.at[pl.ds(0, 16)]],
                                 rows_a.at[pl.ds(0, 16)], sem_a).wait()
                sub(n128 * EW + i * 16, 16, pb_a, rows_a)
                return 0
            lax.fori_loop(0, ((ln + 15) >> 4) - (n128 << 3), rem, 0)

            pltpu.sync_copy(acc.at[pl.ds(0, 512)],
                            out_ref.at[pl.ds(base, 512)])


def _segmax(gpk, hp_pad, loT, lnT):
    mesh = plsc.VectorSubcoreMesh(core_axis_name="c", subcore_axis_name="s")
    f = pl.kernel(
        _smax_body,
        out_type=jax.ShapeDtypeStruct((NPAD2, H2), jnp.float32),
        mesh=mesh,
        compiler_params=pltpu.CompilerParams(needs_layout_passes=False),
        scratch_types=[
            pltpu.VMEM((EW,), jnp.int32),
            pltpu.VMEM((EW,), jnp.int32),
            pltpu.VMEM((EW + 16,), jnp.int32),
            pltpu.VMEM((EW + 16,), jnp.int32),
            pltpu.VMEM((LANES,), jnp.int32),
            pltpu.VMEM((LANES,), jnp.int32),
            pltpu.VMEM((EW, H2), jnp.float32),
            pltpu.VMEM((EW, H2), jnp.float32),
            pltpu.VMEM((521, H2), jnp.float32),
            pltpu.SemaphoreType.DMA,
            pltpu.SemaphoreType.DMA,
        ],
    )
    return f(gpk, hp_pad, loT, lnT)


def _group_edges(ei2):
    cnt = _bucket_count(ei2)[:, :NB]
    tot = jnp.sum(cnt, axis=0)
    tot16 = (tot + 15) & ~15
    bst = jnp.concatenate([jnp.zeros((1,), jnp.int32),
                           jnp.cumsum(tot16, dtype=jnp.int32)])
    excl = jnp.cumsum(cnt, axis=0, dtype=jnp.int32) - cnt
    offs = bst[None, :NB] + excl
    offs = jnp.pad(offs, ((0, 0), (0, NBP - NB)))
    gpk = _bucket_scatter(ei2, offs)
    bstp = jnp.pad(bst[:NB], (0, 224 - NB)).reshape(7, 32).T
    totp = jnp.pad(tot, (0, 224 - NB)).reshape(7, 32).T
    loT = jnp.pad(bstp, ((0, 0), (0, LANES - 7)))
    lnT = jnp.pad(totp, ((0, 0), (0, LANES - 7)))
    return gpk, loT, lnT


def _scale_body(x_ref, cs_ref, cd_ref, xs_ref, di_ref):
    x = x_ref[...]
    xs_ref[...] = x * lax.rsqrt(jnp.maximum(cs_ref[...], 1.0))
    di_ref[...] = lax.rsqrt(jnp.maximum(cd_ref[...], 1.0))


def _scale(x, cs, cd):
    grid = (N // _BLK,)
    return pl.pallas_call(
        _scale_body,
        grid=grid,
        in_specs=[pl.BlockSpec((_BLK, H), lambda i: (i, 0)),
                  pl.BlockSpec((_BLK, 1), lambda i: (i, 0)),
                  pl.BlockSpec((_BLK, 1), lambda i: (i, 0))],
        out_specs=[pl.BlockSpec((_BLK, H), lambda i: (i, 0)),
                   pl.BlockSpec((_BLK, 1), lambda i: (i, 0))],
        out_shape=[jax.ShapeDtypeStruct((N, H), jnp.float32),
                   jax.ShapeDtypeStruct((N, 1), jnp.float32)],
    )(x, cs, cd)


def _pool_body(g_ref, x_ref, out_ref, m_sc, s_sc, a_sc):
    i = pl.program_id(0)
    g = g_ref[...]
    x = x_ref[...]
    bm = jnp.max(g)

    @pl.when(i == 0)
    def _():
        m_sc[0, 0] = bm
        w = jnp.exp(g - bm)
        s_sc[0, 0] = jnp.sum(w)
        a_sc[...] = jnp.sum(w * x, axis=0, keepdims=True)

    @pl.when(i > 0)
    def _():
        m = m_sc[0, 0]
        nm = jnp.maximum(m, bm)
        f = jnp.exp(m - nm)
        w = jnp.exp(g - nm)
        m_sc[0, 0] = nm
        s_sc[0, 0] = s_sc[0, 0] * f + jnp.sum(w)
        a_sc[...] = a_sc[...] * f + jnp.sum(w * x, axis=0, keepdims=True)

    @pl.when(i == pl.num_programs(0) - 1)
    def _():
        out_ref[...] = a_sc[...] / s_sc[0, 0]


def _pool(gate, x):
    grid = (N // _BLK,)
    return pl.pallas_call(
        _pool_body,
        grid=grid,
        in_specs=[pl.BlockSpec((_BLK, 1), lambda i: (i, 0)),
                  pl.BlockSpec((_BLK, H), lambda i: (i, 0))],
        out_specs=pl.BlockSpec((1, H), lambda i: (0, 0)),
        out_shape=jax.ShapeDtypeStruct((1, H), jnp.float32),
        scratch_shapes=[pltpu.SMEM((1, 1), jnp.float32),
                        pltpu.SMEM((1, 1), jnp.float32),
                        pltpu.VMEM((1, H), jnp.float32)],
    )(gate, x)


def _mid_body(agg_ref, di_ref, x_ref, pool_ref, wg_ref, bg_ref, g_ref, b_ref,
              z_ref):
    agg = agg_ref[...] * di_ref[...]
    t = jnp.dot(agg, wg_ref[...], preferred_element_type=jnp.float32)
    gcn = _ln(t + bg_ref[...], g_ref[...], b_ref[...])
    x = x_ref[...]
    z_ref[...] = jnp.concatenate([gcn - x, pool_ref[...] - x], axis=1)


def _mid(agg, di, x, pool, p):
    grid = (N // _BLK,)
    full = lambda shape: pl.BlockSpec(shape, lambda i: (0, 0))
    return pl.pallas_call(
        _mid_body,
        grid=grid,
        in_specs=[pl.BlockSpec((_BLK, H), lambda i: (i, 0)),
                  pl.BlockSpec((_BLK, 1), lambda i: (i, 0)),
                  pl.BlockSpec((_BLK, H), lambda i: (i, 0)),
                  full((1, H)), full((H, H)), full((1, H)),
                  full((1, H)), full((1, H))],
        out_specs=pl.BlockSpec((_BLK, H2), lambda i: (i, 0)),
        out_shape=jax.ShapeDtypeStruct((N, H2), jnp.float32),
    )(agg, di, x, pool, p['W_gcn'], p['b_gcn'][None],
      p['ln_gcn_g'][None], p['ln_gcn_b'][None])


def _sage_pre_body(hh_ref, wp_ref, bp_ref, hp_ref):
    t = jnp.dot(hh_ref[...], wp_ref[...], preferred_element_type=jnp.float32)
    hp_ref[...] = jax.nn.relu(t + bp_ref[...])


def _sage_pre(hh, lp):
    grid = (N // _BLK,)
    full = lambda shape: pl.BlockSpec(shape, lambda i: (0, 0))
    return pl.pallas_call(
        _sage_pre_body,
        grid=grid,
        in_specs=[pl.BlockSpec((_BLK, H2), lambda i: (i, 0)),
                  full((H2, H2)), full((1, H2))],
        out_specs=pl.BlockSpec((_BLK, H2), lambda i: (i, 0)),
        out_shape=jax.ShapeDtypeStruct((N, H2), jnp.float32),
    )(hh, lp['Wp'], lp['bp'][None])


def _sage_post_body(hh_ref, ng_ref, ws_ref, wn_ref, bs_ref, g_ref, b_ref,
                    o_ref):
    t = (jnp.dot(hh_ref[...], ws_ref[...], preferred_element_type=jnp.float32)
         + jnp.dot(ng_ref[...], wn_ref[...], preferred_element_type=jnp.float32)
         + bs_ref[...])
    o_ref[...] = jax.nn.relu(_ln(t, g_ref[...], b_ref[...]))


def _sage_post(hh, neigh, lp):
    grid = (N // _BLK,)
    full = lambda shape: pl.BlockSpec(shape, lambda i: (0, 0))
    return pl.pallas_call(
        _sage_post_body,
        grid=grid,
        in_specs=[pl.BlockSpec((_BLK, H2), lambda i: (i, 0)),
                  pl.BlockSpec((_BLK, H2), lambda i: (i, 0)),
                  full((H2, H2)), full((H2, H2)), full((1, H2)),
                  full((1, H2)), full((1, H2))],
        out_specs=pl.BlockSpec((_BLK, H2), lambda i: (i, 0)),
        out_shape=jax.ShapeDtypeStruct((N, H2), jnp.float32),
    )(hh, neigh, lp['Ws'], lp['Wn'], lp['bs'][None],
      lp['ln_g'][None], lp['ln_b'][None])


def _final_body(hh_ref, ng_ref, ws_ref, wn_ref, bs_ref, g_ref, b_ref,
                wo_ref, bo_ref, sc_ref):
    t = (jnp.dot(hh_ref[...], ws_ref[...], preferred_element_type=jnp.float32)
         + jnp.dot(ng_ref[...], wn_ref[...], preferred_element_type=jnp.float32)
         + bs_ref[...])
    t = jax.nn.relu(_ln(t, g_ref[...], b_ref[...]))
    sc_ref[...] = jnp.dot(t, wo_ref[...],
                          preferred_element_type=jnp.float32) + bo_ref[...]


def _final(hh, neigh, lp, p):
    grid = (N // _BLK,)
    full = lambda shape: pl.BlockSpec(shape, lambda i: (0, 0))
    return pl.pallas_call(
        _final_body,
        grid=grid,
        in_specs=[pl.BlockSpec((_BLK, H2), lambda i: (i, 0)),
                  pl.BlockSpec((_BLK, H2), lambda i: (i, 0)),
                  full((H2, H2)), full((H2, H2)), full((1, H2)),
                  full((1, H2)), full((1, H2)),
                  full((H2, OUT)), full((1, OUT))],
        out_specs=pl.BlockSpec((_BLK, OUT), lambda i: (i, 0)),
        out_shape=jax.ShapeDtypeStruct((N, OUT), jnp.float32),
    )(hh, neigh, lp['Ws'], lp['Wn'], lp['bs'][None],
      lp['ln_g'][None], lp['ln_b'][None], p['W_out'], p['b_out'][None])


def kernel(h, params, edge_index):
    p = params
    ei2 = jnp.concatenate(
        [edge_index, jnp.full((2, E_PAD - E), N, jnp.int32)], axis=1
    ).reshape(2, ROWS, EW)

    cnt_src, cnt_dst = _degrees(ei2)
    x, gate = _pre_chain(h, p)
    xs, di = _scale(x, cnt_src[:N, None], cnt_dst[:N, None])

    xs_pad = jnp.pad(xs, ((0, N_PAD - N), (0, 0)))
    xs_chunks = [xs_pad[:, 16 * f:16 * f + 16] for f in range(4)]
    agg4 = _gcn_agg(xs_chunks, ei2)
    agg = jnp.concatenate([a[:N] for a in agg4], axis=1)

    pool = _pool(gate, x)
    Z = _mid(agg, di, x, pool, p)

    hh = Z
    gpk, loT, lnT = _group_edges(ei2)
    for li, lp in enumerate(p['layers']):
        hp = _sage_pre(hh, lp)
        hp_pad = jnp.pad(hp, ((0, NPAD2 - N), (0, 0)))
        neigh = _segmax(gpk, hp_pad, loT, lnT)[:N]
        if li == 0:
            hh = _sage_post(hh, neigh, lp)
        else:
            score = _final(hh, neigh, lp, p)
    return score, Z


# speedup vs baseline: 2.9335x; 1.0078x over previous
import functools

import jax
import jax.numpy as jnp
from jax import lax
from jax.experimental import pallas as pl
from jax.experimental.pallas import tpu as pltpu
from jax.experimental.pallas import tpu_sc as plsc

N = 100000
E = 1600000
DIN = 17
H = 64
H2 = 128
OUT = 2
NEG = 0.05
EPS = 1e-5

NC, NS, LANES = 2, 16, 16
N_PAD = 100096
NPT = N_PAD // NS
ZR = NPT // 8
EW = 128
ROWS = 12544
E_PAD = ROWS * EW
RPT = ROWS // NS
KB = 4
NBLK = RPT // KB
CZ = 391
NCZ = NPT // CZ

_BLK = 4000


def _ln(x, g, b):
    m = jnp.mean(x, axis=-1, keepdims=True)
    v = jnp.var(x, axis=-1, keepdims=True)
    return (x - m) / jnp.sqrt(v + EPS) * g + b


def _pre_body(h_ref, win_ref, bin_ref, wt1_ref, bt1_ref, wt2_ref, bt2_ref,
              wg_ref, bg_ref, x_ref, g_ref):
    x = jnp.dot(h_ref[...], win_ref[...], preferred_element_type=jnp.float32)
    x = x + bin_ref[...]
    x = jnp.dot(x, wt1_ref[...], preferred_element_type=jnp.float32) + bt1_ref[...]
    x = jnp.where(x >= 0, x, NEG * x)
    x = jnp.dot(x, wt2_ref[...], preferred_element_type=jnp.float32) + bt2_ref[...]
    x_ref[...] = x
    g_ref[...] = jnp.dot(x, wg_ref[...], preferred_element_type=jnp.float32) + bg_ref[...]


def _pre_chain(h, p):
    grid = (N // _BLK,)
    full = lambda shape: pl.BlockSpec(shape, lambda i: (0, 0))
    return pl.pallas_call(
        _pre_body,
        grid=grid,
        in_specs=[
            pl.BlockSpec((_BLK, DIN), lambda i: (i, 0)),
            full((DIN, H)), full((1, H)),
            full((H, H)), full((1, H)),
            full((H, H)), full((1, H)),
            full((H, 1)), full((1, 1)),
        ],
        out_specs=[
            pl.BlockSpec((_BLK, H), lambda i: (i, 0)),
            pl.BlockSpec((_BLK, 1), lambda i: (i, 0)),
        ],
        out_shape=[
            jax.ShapeDtypeStruct((N, H), jnp.float32),
            jax.ShapeDtypeStruct((N, 1), jnp.float32),
        ],
    )(h, p['W_in'], p['b_in'][None], p['W_t1'], p['b_t1'][None],
      p['W_t2'], p['b_t2'][None], p['W_gate'], p['b_gate'][None])


def _deg_body(ei_ref, out_src, out_dst, idx_v, ones_v, zbuf, shared):
    c = lax.axis_index("c")
    s = lax.axis_index("s")
    for k in range(EW // LANES):
        ones_v[pl.ds(LANES * k, LANES)] = jnp.ones((LANES,), jnp.float32)

    def zb(i, _):
        zbuf[pl.ds(i * LANES, LANES)] = jnp.zeros((LANES,), jnp.float32)
        return 0
    lax.fori_loop(0, NPT // LANES, zb, 0)
    pltpu.sync_copy(zbuf, shared.at[pl.ds(s * NPT, NPT)])
    plsc.subcore_barrier()

    for cs in range(NC):
        @pl.when(c == cs)
        def _():
            def blk(b, _):
                base = s * RPT + b * KB
                pltpu.sync_copy(ei_ref.at[cs, pl.ds(base, KB)], idx_v)
                for j in range(KB):
                    pltpu.sync_copy(ones_v, shared.at[idx_v.at[j]], add=True)
                return 0
            lax.fori_loop(0, NBLK, blk, 0)
            plsc.subcore_barrier()
            out = out_src if cs == 0 else out_dst
            pltpu.sync_copy(shared.at[pl.ds(s * NPT, NPT)], zbuf)
            pltpu.sync_copy(zbuf, out.at[pl.ds(s * NPT, NPT)])


def _degrees(ei2):
    mesh = plsc.VectorSubcoreMesh(core_axis_name="c", subcore_axis_name="s")
    f = pl.kernel(
        _deg_body,
        out_type=[jax.ShapeDtypeStruct((N_PAD,), jnp.float32),
                  jax.ShapeDtypeStruct((N_PAD,), jnp.float32)],
        mesh=mesh,
        scratch_types=[
            pltpu.VMEM((KB, EW), jnp.int32),
            pltpu.VMEM((EW,), jnp.float32),
            pltpu.VMEM((NPT,), jnp.float32),
            pltpu.VMEM_SHARED((N_PAD,), jnp.float32),
        ],
    )
    return f(ei2)


def _agg_body(xs0, xs1, xs2, xs3, ei_ref, o0, o1, o2, o3,
              sidx, didx, rows_v, zbuf2, bounce, shared2, sem, sem2):
    c = lax.axis_index("c")
    s = lax.axis_index("s")
    xs_refs = (xs0, xs1, xs2, xs3)
    out_refs = (o0, o1, o2, o3)

    def zb(i, _):
        zbuf2[i, :] = jnp.zeros((LANES,), jnp.float32)
        return 0
    lax.fori_loop(0, CZ, zb, 0)

    for cs in range(NC):
        @pl.when(c == cs)
        def _():
            for cc in range(2):
                fch = cs * 2 + cc
                for k in range(NCZ):
                    pltpu.sync_copy(
                        zbuf2, shared2.at[pl.ds(s * NPT + k * CZ, CZ)])
                plsc.subcore_barrier()

                def blk(b, _):
                    base = s * RPT + b * KB
                    pltpu.sync_copy(ei_ref.at[0, pl.ds(base, KB)], sidx)
                    pltpu.sync_copy(ei_ref.at[1, pl.ds(base, KB)], didx)
                    cps = [pltpu.async_copy(xs_refs[fch].at[sidx.at[j]],
                                            rows_v.at[j], sem)
                           for j in range(KB)]
                    for cp in cps:
                        cp.wait()
                    cps2 = [pltpu.async_copy(rows_v.at[j],
                                             shared2.at[didx.at[j]], sem2,
                                             add=True)
                            for j in range(KB)]
                    for cp in cps2:
                        cp.wait()
                    return 0
                lax.fori_loop(0, NBLK, blk, 0)
                plsc.subcore_barrier()
                for k in range(NCZ):
                    pltpu.sync_copy(
                        shared2.at[pl.ds(s * NPT + k * CZ, CZ)], bounce)
                    pltpu.sync_copy(
                        bounce, out_refs[fch].at[pl.ds(s * NPT + k * CZ, CZ)])
                plsc.subcore_barrier()


def _gcn_agg(xs_chunks, ei2):
    mesh = plsc.VectorSubcoreMesh(core_axis_name="c", subcore_axis_name="s")
    f = pl.kernel(
        _agg_body,
        out_type=[jax.ShapeDtypeStruct((N_PAD, LANES), jnp.float32)] * 4,
        mesh=mesh,
        compiler_params=pltpu.CompilerParams(use_tc_tiling_on_sc=False),
        scratch_types=[
            pltpu.VMEM((KB, EW), jnp.int32),
            pltpu.VMEM((KB, EW), jnp.int32),
            pltpu.VMEM((KB, EW, LANES), jnp.float32),
            pltpu.VMEM((CZ, LANES), jnp.float32),
            pltpu.VMEM((CZ, LANES), jnp.float32),
            pltpu.VMEM_SHARED((N_PAD, LANES), jnp.float32),
            pltpu.SemaphoreType.DMA,
            pltpu.SemaphoreType.DMA,
        ],
    )
    return f(*xs_chunks, ei2)


NB = 196
NBP = 224
BSP = 224
NPAD2 = NB * 512
KB2 = 8
RPT2 = ROWS // (NC * NS)
NBLK2 = RPT2 // KB2
GE = E_PAD + NB * 16

_IOTA = lambda: lax.iota(jnp.int32, LANES)


def _vgather(x, idx):
    return lax.gather(
        x, idx[:, None],
        lax.GatherDimensionNumbers(offset_dims=(), collapsed_slice_dims=(0,),
                                   start_index_map=(0,)),
        (1,), mode=lax.GatherScatterMode.PROMISE_IN_BOUNDS)


def _runs(bvec):
    it = _IOTA()
    bs, perm = plsc.sort_key_val(bvec, it)
    prev = _vgather(bs, jnp.maximum(it - 1, 0))
    nxt = _vgather(bs, jnp.minimum(it + 1, LANES - 1))
    isstart = (it == 0) | (bs != prev)
    isend = (it == LANES - 1) | (bs != nxt)
    ri = it - plsc.cummax(jnp.where(isstart, it, 0))
    return bs, perm, ri, isend


def _b1_body(ei_ref, cnt_out, dbuf, cnt_v):
    c = lax.axis_index("c")
    s = lax.axis_index("s")
    wid = c * NS + s
    for k in range(NBP // LANES):
        cnt_v[pl.ds(k * LANES, LANES)] = jnp.zeros((LANES,), jnp.int32)

    def blk(i, _):
        pltpu.sync_copy(ei_ref.at[1, pl.ds(wid * RPT2 + i * KB2, KB2)], dbuf)
        for j in range(KB2):
            for k in range(EW // LANES):
                b = dbuf[j, pl.ds(k * LANES, LANES)] >> 9
                bs, _p, ri, isend = _runs(b)
                cnts = plsc.load_gather(cnt_v, [bs])
                plsc.store_scatter(cnt_v, [bs], cnts + ri + 1, mask=isend)
        return 0
    lax.fori_loop(0, NBLK2, blk, 0)
    pltpu.sync_copy(cnt_v, cnt_out.at[wid])


def _bucket_count(ei2):
    mesh = plsc.VectorSubcoreMesh(core_axis_name="c", subcore_axis_name="s")
    f = pl.kernel(
        _b1_body,
        out_type=jax.ShapeDtypeStruct((32, NBP), jnp.int32),
        mesh=mesh,
        compiler_params=pltpu.CompilerParams(needs_layout_passes=False),
        scratch_types=[
            pltpu.VMEM((KB2, EW), jnp.int32),
            pltpu.VMEM((NBP,), jnp.int32),
        ],
    )
    return f(ei2)


def _b2_body(ei_ref, offs_ref, g_pk,
             sbuf, dbuf, posb, pvb, pos_v, sem):
    c = lax.axis_index("c")
    s = lax.axis_index("s")
    wid = c * NS + s
    pltpu.sync_copy(offs_ref.at[wid], pos_v)

    def blk(i, _):
        pltpu.sync_copy(ei_ref.at[0, pl.ds(wid * RPT2 + i * KB2, KB2)], sbuf)
        pltpu.sync_copy(ei_ref.at[1, pl.ds(wid * RPT2 + i * KB2, KB2)], dbuf)
        for j in range(KB2):
            for k in range(EW // LANES):
                sv = sbuf[j, pl.ds(k * LANES, LANES)]
                dv = dbuf[j, pl.ds(k * LANES, LANES)]
                b = dv >> 9
                pv = (sv << 9) | (dv & 511)
                it = _IOTA()
                bs, pvs = plsc.sort_key_val(b, pv)
                prev = _vgather(bs, jnp.maximum(it - 1, 0))
                nxt = _vgather(bs, jnp.minimum(it + 1, LANES - 1))
                isstart = (it == 0) | (bs != prev)
                isend = (it == LANES - 1) | (bs != nxt)
                ri = it - plsc.cummax(jnp.where(isstart, it, 0))
                curs = plsc.load_gather(pos_v, [bs])
                pos = curs + ri
                plsc.store_scatter(pos_v, [bs], pos + 1, mask=isend)
                posb[j, pl.ds(k * LANES, LANES)] = pos
                pvb[j, pl.ds(k * LANES, LANES)] = pvs
        cps = [pltpu.async_copy(pvb.at[j], g_pk.at[posb.at[j]], sem)
               for j in range(KB2)]
        for cp in cps:
            cp.wait()
        return 0
    lax.fori_loop(0, NBLK2, blk, 0)


def _bucket_scatter(ei2, offs):
    mesh = plsc.VectorSubcoreMesh(core_axis_name="c", subcore_axis_name="s")
    f = pl.kernel(
        _b2_body,
        out_type=jax.ShapeDtypeStruct((GE,), jnp.int32),
        mesh=mesh,
        compiler_params=pltpu.CompilerParams(needs_layout_passes=False),
        scratch_types=[
            pltpu.VMEM((KB2, EW), jnp.int32),
            pltpu.VMEM((KB2, EW), jnp.int32),
            pltpu.VMEM((KB2, EW), jnp.int32),
            pltpu.VMEM((KB2, EW), jnp.int32),
            pltpu.VMEM((NBP,), jnp.int32),
            pltpu.SemaphoreType.DMA,
        ],
    )
    return f(ei2, offs)


def _smax_body(gpk_ref, hp_ref, lo_ref, ln_ref, out_ref,
               sb_a, sb_b, pb_a, pb_b, lov, lnv, rows_a, rows_b, acc,
               sem_a, sem_b):
    c = lax.axis_index("c")
    s = lax.axis_index("s")
    wid = c * NS + s
    pltpu.sync_copy(lo_ref.at[wid], lov)
    pltpu.sync_copy(ln_ref.at[wid], lnv)
    lo_all = lov[pl.ds(0, LANES)]
    ln_all = lnv[pl.ds(0, LANES)]

    for k in range(7):
        b = k * 32 + wid

        @pl.when(b < NB)
        def _():
            lo = lo_all[k]
            ln = ln_all[k]
            base = b * 512

            def za(i, _):
                for f8 in range(8):
                    acc[i, pl.ds(f8 * LANES, LANES)] = jnp.zeros(
                        (LANES,), jnp.float32)
                return 0
            lax.fori_loop(0, 520, za, 0)

            n128 = ln >> 7

            def start(i, sb, pb, rows, sem):
                p = pl.multiple_of(lo + i * EW, 16)
                pltpu.sync_copy(gpk_ref.at[pl.ds(p, EW)], pb.at[pl.ds(0, EW)])
                for q in range(EW // LANES):
                    v = pb[pl.ds(q * LANES, LANES)]
                    sb[pl.ds(q * LANES, LANES)] = jnp.clip(v >> 9, 0, N)
                pltpu.async_copy(hp_ref.at[sb], rows, sem)

            def drain(rows, sem):
                pltpu.make_async_copy(
                    hp_ref.at[pl.ds(0, EW)], rows, sem).wait()

            def sub(off, cnt, pb, rows):
                def grp(g, _):
                    dv16 = pb[pl.ds(g * 8, 16)]
                    for j in range(8):
                        e = g * 8 + j
                        dl = dv16[j] & 511
                        dl = jnp.where(off + e < ln, dl, 512)
                        for f8 in range(8):
                            acc[dl, pl.ds(f8 * LANES, LANES)] = jnp.maximum(
                                acc[dl, pl.ds(f8 * LANES, LANES)],
                                rows[e, pl.ds(f8 * LANES, LANES)])
                    return 0
                lax.fori_loop(0, cnt >> 3, grp, 0)

            @pl.when(n128 > 0)
            def _():
                start(0, sb_a, pb_a, rows_a, sem_a)

            def big(i, _):
                even = (i & 1) == 0

                @pl.when(even)
                def _():
                    @pl.when(i + 1 < n128)
                    def _():
                        start(i + 1, sb_b, pb_b, rows_b, sem_b)
                    drain(rows_a, sem_a)
                    sub(i * EW, EW, pb_a, rows_a)

                @pl.when(jnp.logical_not(even))
                def _():
                    @pl.when(i + 1 < n128)
                    def _():
                        start(i + 1, sb_a, pb_a, rows_a, sem_a)
                    drain(rows_b, sem_b)
                    sub(i * EW, EW, pb_b, rows_b)
                return 0
            lax.fori_loop(0, n128, big, 0)

            def rem(i, _):
                p = pl.multiple_of(lo + n128 * EW + i * 16, 16)
                pltpu.sync_copy(gpk_ref.at[pl.ds(p, 16)], pb_a.at[pl.ds(0, 16)])
                v = pb_a[pl.ds(0, LANES)]
                sb_a[pl.ds(0, LANES)] = jnp.clip(v >> 9, 0, N)
                pltpu.async_copy(hp_ref.at[sb_a.at[pl.ds(0, 16)]],
                                 rows_a.at[pl.ds(0, 16)], sem_a).wait()
                sub(n128 * EW + i * 16, 16, pb_a, rows_a)
                return 0
            lax.fori_loop(0, ((ln + 15) >> 4) - (n128 << 3), rem, 0)

            pltpu.sync_copy(acc.at[pl.ds(0, 512)],
                            out_ref.at[pl.ds(base, 512)])


def _segmax(gpk, hp_pad, loT, lnT):
    mesh = plsc.VectorSubcoreMesh(core_axis_name="c", subcore_axis_name="s")
    f = pl.kernel(
        _smax_body,
        out_type=jax.ShapeDtypeStruct((NPAD2, H2), jnp.float32),
        mesh=mesh,
        compiler_params=pltpu.CompilerParams(needs_layout_passes=False),
        scratch_types=[
            pltpu.VMEM((EW,), jnp.int32),
            pltpu.VMEM((EW,), jnp.int32),
            pltpu.VMEM((EW + 16,), jnp.int32),
            pltpu.VMEM((EW + 16,), jnp.int32),
            pltpu.VMEM((LANES,), jnp.int32),
            pltpu.VMEM((LANES,), jnp.int32),
            pltpu.VMEM((EW, H2), jnp.float32),
            pltpu.VMEM((EW, H2), jnp.float32),
            pltpu.VMEM((521, H2), jnp.float32),
            pltpu.SemaphoreType.DMA,
            pltpu.SemaphoreType.DMA,
        ],
    )
    return f(gpk, hp_pad, loT, lnT)


def _group_edges(ei2):
    cnt = _bucket_count(ei2)[:, :NB]
    tot = jnp.sum(cnt, axis=0)
    tot16 = (tot + 15) & ~15
    bst = jnp.concatenate([jnp.zeros((1,), jnp.int32),
                           jnp.cumsum(tot16, dtype=jnp.int32)])
    excl = jnp.cumsum(cnt, axis=0, dtype=jnp.int32) - cnt
    offs = bst[None, :NB] + excl
    offs = jnp.pad(offs, ((0, 0), (0, NBP - NB)))
    gpk = _bucket_scatter(ei2, offs)
    bstp = jnp.pad(bst[:NB], (0, 224 - NB)).reshape(7, 32).T
    totp = jnp.pad(tot, (0, 224 - NB)).reshape(7, 32).T
    loT = jnp.pad(bstp, ((0, 0), (0, LANES - 7)))
    lnT = jnp.pad(totp, ((0, 0), (0, LANES - 7)))
    return gpk, loT, lnT


def _scale_body(x_ref, cs_ref, cd_ref, xs_ref, di_ref):
    x = x_ref[...]
    xs_ref[...] = x * lax.rsqrt(jnp.maximum(cs_ref[...], 1.0))
    di_ref[...] = lax.rsqrt(jnp.maximum(cd_ref[...], 1.0))


def _scale(x, cs, cd):
    grid = (N // _BLK,)
    return pl.pallas_call(
        _scale_body,
        grid=grid,
        in_specs=[pl.BlockSpec((_BLK, H), lambda i: (i, 0)),
                  pl.BlockSpec((_BLK, 1), lambda i: (i, 0)),
                  pl.BlockSpec((_BLK, 1), lambda i: (i, 0))],
        out_specs=[pl.BlockSpec((_BLK, H), lambda i: (i, 0)),
                   pl.BlockSpec((_BLK, 1), lambda i: (i, 0))],
        out_shape=[jax.ShapeDtypeStruct((N, H), jnp.float32),
                   jax.ShapeDtypeStruct((N, 1), jnp.float32)],
    )(x, cs, cd)


def _pool_body(g_ref, x_ref, out_ref, m_sc, s_sc, a_sc):
    i = pl.program_id(0)
    g = g_ref[...]
    x = x_ref[...]
    bm = jnp.max(g)

    @pl.when(i == 0)
    def _():
        m_sc[0, 0] = bm
        w = jnp.exp(g - bm)
        s_sc[0, 0] = jnp.sum(w)
        a_sc[...] = jnp.sum(w * x, axis=0, keepdims=True)

    @pl.when(i > 0)
    def _():
        m = m_sc[0, 0]
        nm = jnp.maximum(m, bm)
        f = jnp.exp(m - nm)
        w = jnp.exp(g - nm)
        m_sc[0, 0] = nm
        s_sc[0, 0] = s_sc[0, 0] * f + jnp.sum(w)
        a_sc[...] = a_sc[...] * f + jnp.sum(w * x, axis=0, keepdims=True)

    @pl.when(i == pl.num_programs(0) - 1)
    def _():
        out_ref[...] = a_sc[...] / s_sc[0, 0]


def _pool(gate, x):
    grid = (N // _BLK,)
    return pl.pallas_call(
        _pool_body,
        grid=grid,
        in_specs=[pl.BlockSpec((_BLK, 1), lambda i: (i, 0)),
                  pl.BlockSpec((_BLK, H), lambda i: (i, 0))],
        out_specs=pl.BlockSpec((1, H), lambda i: (0, 0)),
        out_shape=jax.ShapeDtypeStruct((1, H), jnp.float32),
        scratch_shapes=[pltpu.SMEM((1, 1), jnp.float32),
                        pltpu.SMEM((1, 1), jnp.float32),
                        pltpu.VMEM((1, H), jnp.float32)],
    )(gate, x)


def _mid_body(agg_ref, di_ref, x_ref, pool_ref, wg_ref, bg_ref, g_ref, b_ref,
              z_ref):
    agg = agg_ref[...] * di_ref[...]
    t = jnp.dot(agg, wg_ref[...], preferred_element_type=jnp.float32)
    gcn = _ln(t + bg_ref[...], g_ref[...], b_ref[...])
    x = x_ref[...]
    z_ref[...] = jnp.concatenate([gcn - x, pool_ref[...] - x], axis=1)


def _mid(agg, di, x, pool, p):
    grid = (N // _BLK,)
    full = lambda shape: pl.BlockSpec(shape, lambda i: (0, 0))
    return pl.pallas_call(
        _mid_body,
        grid=grid,
        in_specs=[pl.BlockSpec((_BLK, H), lambda i: (i, 0)),
                  pl.BlockSpec((_BLK, 1), lambda i: (i, 0)),
                  pl.BlockSpec((_BLK, H), lambda i: (i, 0)),
                  full((1, H)), full((H, H)), full((1, H)),
                  full((1, H)), full((1, H))],
        out_specs=pl.BlockSpec((_BLK, H2), lambda i: (i, 0)),
        out_shape=jax.ShapeDtypeStruct((N, H2), jnp.float32),
    )(agg, di, x, pool, p['W_gcn'], p['b_gcn'][None],
      p['ln_gcn_g'][None], p['ln_gcn_b'][None])


def _sage_pre_body(hh_ref, wp_ref, bp_ref, hp_ref):
    t = jnp.dot(hh_ref[...], wp_ref[...], preferred_element_type=jnp.float32)
    hp_ref[...] = jax.nn.relu(t + bp_ref[...])


def _sage_pre(hh, lp):
    grid = (N // _BLK,)
    full = lambda shape: pl.BlockSpec(shape, lambda i: (0, 0))
    return pl.pallas_call(
        _sage_pre_body,
        grid=grid,
        in_specs=[pl.BlockSpec((_BLK, H2), lambda i: (i, 0)),
                  full((H2, H2)), full((1, H2))],
        out_specs=pl.BlockSpec((_BLK, H2), lambda i: (i, 0)),
        out_shape=jax.ShapeDtypeStruct((N, H2), jnp.float32),
    )(hh, lp['Wp'], lp['bp'][None])


def _sage_post_body(hh_ref, ng_ref, ws_ref, wn_ref, bs_ref, g_ref, b_ref,
                    o_ref):
    t = (jnp.dot(hh_ref[...], ws_ref[...], preferred_element_type=jnp.float32)
         + jnp.dot(ng_ref[...], wn_ref[...], preferred_element_type=jnp.float32)
         + bs_ref[...])
    o_ref[...] = jax.nn.relu(_ln(t, g_ref[...], b_ref[...]))


def _sage_post(hh, neigh, lp):
    grid = (N // _BLK,)
    full = lambda shape: pl.BlockSpec(shape, lambda i: (0, 0))
    return pl.pallas_call(
        _sage_post_body,
        grid=grid,
        in_specs=[pl.BlockSpec((_BLK, H2), lambda i: (i, 0)),
                  pl.BlockSpec((_BLK, H2), lambda i: (i, 0)),
                  full((H2, H2)), full((H2, H2)), full((1, H2)),
                  full((1, H2)), full((1, H2))],
        out_specs=pl.BlockSpec((_BLK, H2), lambda i: (i, 0)),
        out_shape=jax.ShapeDtypeStruct((N, H2), jnp.float32),
    )(hh, neigh, lp['Ws'], lp['Wn'], lp['bs'][None],
      lp['ln_g'][None], lp['ln_b'][None])


def _final_body(hh_ref, ng_ref, ws_ref, wn_ref, bs_ref, g_ref, b_ref,
                wo_ref, bo_ref, sc_ref):
    t = (jnp.dot(hh_ref[...], ws_ref[...], preferred_element_type=jnp.float32)
         + jnp.dot(ng_ref[...], wn_ref[...], preferred_element_type=jnp.float32)
         + bs_ref[...])
    t = jax.nn.relu(_ln(t, g_ref[...], b_ref[...]))
    sc_ref[...] = jnp.dot(t, wo_ref[...],
                          preferred_element_type=jnp.float32) + bo_ref[...]


def _final(hh, neigh, lp, p):
    grid = (N // _BLK,)
    full = lambda shape: pl.BlockSpec(shape, lambda i: (0, 0))
    return pl.pallas_call(
        _final_body,
        grid=grid,
        in_specs=[pl.BlockSpec((_BLK, H2), lambda i: (i, 0)),
                  pl.BlockSpec((_BLK, H2), lambda i: (i, 0)),
                  full((H2, H2)), full((H2, H2)), full((1, H2)),
                  full((1, H2)), full((1, H2)),
                  full((H2, OUT)), full((1, OUT))],
        out_specs=pl.BlockSpec((_BLK, OUT), lambda i: (i, 0)),
        out_shape=jax.ShapeDtypeStruct((N, OUT), jnp.float32),
    )(hh, neigh, lp['Ws'], lp['Wn'], lp['bs'][None],
      lp['ln_g'][None], lp['ln_b'][None], p['W_out'], p['b_out'][None])


def kernel(h, params, edge_index):
    p = params
    ei2 = jnp.concatenate(
        [edge_index, jnp.full((2, E_PAD - E), N, jnp.int32)], axis=1
    ).reshape(2, ROWS, EW)

    cnt_src, cnt_dst = _degrees(ei2)
    x, gate = _pre_chain(h, p)
    xs, di = _scale(x, cnt_src[:N, None], cnt_dst[:N, None])

    xs_pad = jnp.pad(xs, ((0, N_PAD - N), (0, 0)))
    xs_chunks = [xs_pad[:, 16 * f:16 * f + 16] for f in range(4)]
    agg4 = _gcn_agg(xs_chunks, ei2)
    agg = jnp.concatenate([a[:N] for a in agg4], axis=1)

    pool = _pool(gate, x)
    Z = _mid(agg, di, x, pool, p)

    hh = Z
    gpk, loT, lnT = _group_edges(ei2)
    for li, lp in enumerate(p['layers']):
        hp = _sage_pre(hh, lp)
        hp_pad = jnp.pad(hp, ((0, NPAD2 - N), (0, 0)))
        neigh = _segmax(gpk, hp_pad, loT, lnT)[:N]
        if li == 0:
            hh = _sage_post(hh, neigh, lp)
        else:
            score = _final(hh, neigh, lp, p)
    return score, Z
